# Initial kernel scaffold; baseline (speedup 1.0000x reference)
import jax
import jax.numpy as jnp
from jax.experimental import pallas as pl


def kernel(x, edge_index, Wq1, Wk1, Wv1, g1, b1, Wq2, Wk2, Wv2, g2, b2, Wq3, Wk3, Wv3):
    raise NotImplementedError("write your pallas kernel here")


# speedup vs baseline: 16.6596x; 16.6596x over previous
import functools
import numpy as np
import jax
import jax.numpy as jnp
from jax import lax
from jax.experimental import pallas as pl
from jax.experimental.pallas import tpu as pltpu
from jax.experimental.pallas import tpu_sc as plsc

N_NODES_C = 10000
N_EDGES_C = 160000
ROW_BLK = 200

SC_NC = 2
SC_NS = 16
SC_NW = SC_NC * SC_NS
NPW = 320
NPAD = SC_NW * NPW
ST_LEN = 336
INST_LEN = NPW * (SC_NW - 1) + ST_LEN


def _lane_gather(x, idx):
    dn = lax.GatherDimensionNumbers(offset_dims=(), collapsed_slice_dims=(0,),
                                    start_index_map=(0,))
    return lax.gather(x, idx[:, None], dn, slice_sizes=(1,),
                      mode=lax.GatherScatterMode.PROMISE_IN_BOUNDS)


def _allsum16(x):
    i = lax.iota(jnp.int32, 16)
    for k in (1, 2, 4, 8):
        x = x + _lane_gather(x, i ^ k)
    return x


def _proj_body(x_ref, wq_ref, wkv_ref, q_ref, kv_ref):
    xb = x_ref[...]
    q_ref[...] = jnp.dot(xb, wq_ref[...], preferred_element_type=jnp.float32)
    kv_ref[...] = jnp.dot(xb, wkv_ref[...], preferred_element_type=jnp.float32)


def _proj1(x, Wq, Wkv):
    n, f = x.shape
    dq, dkv = Wq.shape[1], Wkv.shape[1]
    grid = n // ROW_BLK
    return pl.pallas_call(
        _proj_body,
        grid=(grid,),
        in_specs=[
            pl.BlockSpec((ROW_BLK, f), lambda i: (i, 0)),
            pl.BlockSpec((f, dq), lambda i: (0, 0)),
            pl.BlockSpec((f, dkv), lambda i: (0, 0)),
        ],
        out_specs=[
            pl.BlockSpec((ROW_BLK, dq), lambda i: (i, 0)),
            pl.BlockSpec((ROW_BLK, dkv), lambda i: (i, 0)),
        ],
        out_shape=[
            jax.ShapeDtypeStruct((n, dq), jnp.float32),
            jax.ShapeDtypeStruct((n, dkv), jnp.float32),
        ],
    )(x, Wq, Wkv)


def _ln_proj_body(h_ref, g_ref, b_ref, wq_ref, wkv_ref, q_ref, kv_ref):
    h = h_ref[...]
    mu = jnp.mean(h, axis=-1, keepdims=True)
    var = jnp.mean(jnp.square(h - mu), axis=-1, keepdims=True)
    hn = (h - mu) * lax.rsqrt(var + 1e-5) * g_ref[...] + b_ref[...]
    hn = jnp.maximum(hn, 0.0)
    q_ref[...] = jnp.dot(hn, wq_ref[...], preferred_element_type=jnp.float32)
    kv_ref[...] = jnp.dot(hn, wkv_ref[...], preferred_element_type=jnp.float32)


def _ln_proj(h, g, b, Wq, Wkv):
    n, f = h.shape
    dq, dkv = Wq.shape[1], Wkv.shape[1]
    grid = n // ROW_BLK
    return pl.pallas_call(
        _ln_proj_body,
        grid=(grid,),
        in_specs=[
            pl.BlockSpec((ROW_BLK, f), lambda i: (i, 0)),
            pl.BlockSpec((f,), lambda i: (0,)),
            pl.BlockSpec((f,), lambda i: (0,)),
            pl.BlockSpec((f, dq), lambda i: (0, 0)),
            pl.BlockSpec((f, dkv), lambda i: (0, 0)),
        ],
        out_specs=[
            pl.BlockSpec((ROW_BLK, dq), lambda i: (i, 0)),
            pl.BlockSpec((ROW_BLK, dkv), lambda i: (i, 0)),
        ],
        out_shape=[
            jax.ShapeDtypeStruct((n, dq), jnp.float32),
            jax.ShapeDtypeStruct((n, dkv), jnp.float32),
        ],
    )(h, g, b, Wq, Wkv)


def _ln_proj4_body(h_ref, g_ref, b_ref, wa_ref, wb_ref, wc_ref, wd_ref,
                   oa_ref, ob_ref, oc_ref, od_ref):
    h = h_ref[...]
    mu = jnp.mean(h, axis=-1, keepdims=True)
    var = jnp.mean(jnp.square(h - mu), axis=-1, keepdims=True)
    hn = (h - mu) * lax.rsqrt(var + 1e-5) * g_ref[...] + b_ref[...]
    hn = jnp.maximum(hn, 0.0)
    oa_ref[...] = jnp.dot(hn, wa_ref[...], preferred_element_type=jnp.float32)
    ob_ref[...] = jnp.dot(hn, wb_ref[...], preferred_element_type=jnp.float32)
    oc_ref[...] = jnp.dot(hn, wc_ref[...], preferred_element_type=jnp.float32)
    od_ref[...] = jnp.dot(hn, wd_ref[...], preferred_element_type=jnp.float32)


def _ln_proj4(h, g, b, Wa, Wb, Wc, Wd):
    n, f = h.shape
    dims = [W.shape[1] for W in (Wa, Wb, Wc, Wd)]
    grid = n // ROW_BLK
    return pl.pallas_call(
        _ln_proj4_body,
        grid=(grid,),
        in_specs=[
            pl.BlockSpec((ROW_BLK, f), lambda i: (i, 0)),
            pl.BlockSpec((f,), lambda i: (0,)),
            pl.BlockSpec((f,), lambda i: (0,)),
        ] + [pl.BlockSpec((f, d), lambda i: (0, 0)) for d in dims],
        out_specs=[pl.BlockSpec((ROW_BLK, d), lambda i: (i, 0)) for d in dims],
        out_shape=[jax.ShapeDtypeStruct((n, d), jnp.float32) for d in dims],
    )(h, g, b, Wa, Wb, Wc, Wd)


def _seg_attn_sc(q, kv, srcs_pad, inst, heads):
    n, dq = q.shape
    dkv = kv.shape[1]
    dh = dq // heads
    ncq = dh // 16
    scale = 1.0 / np.sqrt(float(dh))
    mesh = plsc.VectorSubcoreMesh(core_axis_name="c", subcore_axis_name="s")

    @functools.partial(
        pl.kernel, mesh=mesh,
        out_type=jax.ShapeDtypeStruct((NPAD, dq), jnp.float32),
        scratch_types=[
            pltpu.VMEM((ST_LEN,), jnp.int32),
            pltpu.VMEM((24,), jnp.int32),
            pltpu.VMEM((16, dq), jnp.float32),
            pltpu.VMEM((16, dkv), jnp.float32),
            pltpu.VMEM((16, dq), jnp.float32),
            pltpu.SemaphoreType.DMA,
            pltpu.SemaphoreType.DMA,
        ],
    )
    def k(q_h, kv_h, srcs_h, inst_h, out_h, st_v, ibuf, qbuf, kvbuf, obuf,
          semq, semkv):
        w = lax.axis_index("s") * SC_NC + lax.axis_index("c")
        base = w * NPW
        pltpu.sync_copy(inst_h.at[pl.ds(pl.multiple_of(base, 8), ST_LEN)], st_v)

        def group(t, _g):
            v0 = base + t * 16
            ids = jnp.minimum(v0 + lax.iota(jnp.int32, 16), n - 1)
            pltpu.async_copy(q_h.at[ids], qbuf, semq).wait()

            def node(l, _n):
                stv = st_v[pl.ds(t * 16 + l, 16)]
                lo = stv[0]
                hi = stv[1]
                cnt = hi - lo
                for jj in range(dq // 16):
                    obuf[l, pl.ds(jj * 16, 16)] = jnp.zeros((16,), jnp.float32)
                nb = (cnt + 15) // 16

                def ebatch(b, carry):
                    dens, s0s = carry
                    a = lo + b * 16
                    a8 = pl.multiple_of((a // 8) * 8, 8)
                    pltpu.sync_copy(srcs_h.at[pl.ds(a8, 24)], ibuf)
                    ids_e = ibuf[pl.ds(a - a8, 16)]
                    pltpu.async_copy(kv_h.at[ids_e], kvbuf, semkv).wait()

                    def edge(l2, carry2):
                        dens2, s0s2 = carry2
                        j = b * 16 + l2
                        new_d, new_s = [], []
                        for h in range(heads):
                            dot = jnp.zeros((16,), jnp.float32)
                            for jj in range(ncq):
                                c0 = h * dh + jj * 16
                                dot = dot + (qbuf[l, pl.ds(c0, 16)]
                                             * kvbuf[l2, pl.ds(c0, 16)])
                            s = _allsum16(dot) * scale
                            s0h = jnp.where(j == 0, s, s0s2[h])
                            wgt = jnp.exp(s - s0h)
                            new_d.append(dens2[h] + wgt)
                            new_s.append(s0h)
                            for jj in range(ncq):
                                c0 = h * dh + jj * 16
                                obuf[l, pl.ds(c0, 16)] = (
                                    obuf[l, pl.ds(c0, 16)]
                                    + wgt * kvbuf[l2, pl.ds(dq + c0, 16)])
                        return tuple(new_d), tuple(new_s)

                    return lax.fori_loop(0, jnp.minimum(16, cnt - b * 16),
                                         edge, (dens, s0s))

                zero = tuple(jnp.zeros((16,), jnp.float32) for _ in range(heads))
                dens, _ = lax.fori_loop(0, nb, ebatch, (zero, zero))
                for h in range(heads):
                    rden = jnp.where(dens[h] > 0.0, 1.0 / dens[h], 0.0)
                    for jj in range(ncq):
                        c0 = h * dh + jj * 16
                        obuf[l, pl.ds(c0, 16)] = obuf[l, pl.ds(c0, 16)] * rden
                return _n

            lax.fori_loop(0, 16, node, 0)
            pltpu.sync_copy(obuf, out_h.at[pl.ds(pl.multiple_of(v0, 8), 16)])
            return _g

        lax.fori_loop(0, NPW // 16, group, 0)

    return k(q, kv, srcs_pad, inst)


def _line_attn_sc(pqA, pqB, pkvA, pkvB, srcs_pad, dsts_pad, perm_pad, inst, outst):
    n, d = pqA.shape
    nd = d // 16
    e_pad = perm_pad.shape[0] - 32
    scale = 1.0 / np.sqrt(float(d))
    KCAP = 64
    mesh = plsc.VectorSubcoreMesh(core_axis_name="c", subcore_axis_name="s")

    @functools.partial(
        pl.kernel, mesh=mesh,
        out_type=jax.ShapeDtypeStruct((e_pad + 16, d), jnp.float32),
        scratch_types=[
            pltpu.VMEM((ST_LEN,), jnp.int32),
            pltpu.VMEM((ST_LEN,), jnp.int32),
            pltpu.VMEM((24,), jnp.int32),
            pltpu.VMEM((24,), jnp.int32),
            pltpu.VMEM((24,), jnp.int32),
            pltpu.VMEM((16, d), jnp.float32),
            pltpu.VMEM((16, 2 * d), jnp.float32),
            pltpu.VMEM((16, d), jnp.float32),
            pltpu.VMEM((KCAP, 2 * d), jnp.float32),
            pltpu.VMEM((16, d), jnp.float32),
            pltpu.VMEM((16 * (nd + 3) * 16,), jnp.float32),
            pltpu.SemaphoreType.DMA,
            pltpu.SemaphoreType.DMA,
        ],
    )
    def k(pqA_h, pqB_h, pkvA_h, pkvB_h, srcs_h, dsts_h, perm_h, inst_h,
          outst_h, out_h, st_in, st_out, sbuf, dbuf, pbuf, abuf, bbuf, qbuf,
          kvc, obuf, state, sem1, sem2):
        w = lax.axis_index("s") * SC_NC + lax.axis_index("c")
        base = w * NPW
        pltpu.sync_copy(inst_h.at[pl.ds(pl.multiple_of(base, 8), ST_LEN)], st_in)
        pltpu.sync_copy(outst_h.at[pl.ds(pl.multiple_of(base, 8), ST_LEN)], st_out)
        stride = (nd + 3) * 16

        def group(t, _g):
            v0 = base + t * 16
            ids = jnp.minimum(v0 + lax.iota(jnp.int32, 16), n - 1)
            pltpu.async_copy(pqA_h.at[ids], abuf, sem1).wait()
            pltpu.async_copy(pkvB_h.at[ids], bbuf, sem1).wait()

            def node(l, _n):
                sti = st_in[pl.ds(t * 16 + l, 16)]
                klo = sti[0]
                kcnt = sti[1] - klo
                sto = st_out[pl.ds(t * 16 + l, 16)]
                jlo = sto[0]
                ocnt = sto[1] - jlo

                @pl.when(ocnt > 0)
                def _():
                    nks = (kcnt + KCAP - 1) // KCAP
                    ntb = (ocnt + 15) // 16

                    def tbatch(tb, _t):
                        j0 = jlo + tb * 16
                        rem = ocnt - tb * 16
                        a8 = pl.multiple_of((j0 // 8) * 8, 8)
                        pltpu.sync_copy(dsts_h.at[pl.ds(a8, 24)], dbuf)
                        tids = dbuf[pl.ds(j0 - a8, 16)]
                        pltpu.async_copy(pqB_h.at[tids], qbuf, sem1).wait()
                        pltpu.sync_copy(perm_h.at[pl.ds(a8, 24)], pbuf)
                        pids = pbuf[pl.ds(j0 - a8, 16)]
                        pids = jnp.where(lax.iota(jnp.int32, 16)
                                         < jnp.minimum(rem, 16), pids, e_pad)
                        nt = jnp.minimum(16, rem)

                        def tinit(l2, _i):
                            cB = jnp.zeros((16,), jnp.float32)
                            for jj in range(nd):
                                qv = (qbuf[l2, pl.ds(jj * 16, 16)]
                                      + abuf[l, pl.ds(jj * 16, 16)])
                                qbuf[l2, pl.ds(jj * 16, 16)] = qv
                                cB = cB + qv * bbuf[l, pl.ds(jj * 16, 16)]
                                state[pl.ds(l2 * stride + jj * 16, 16)] = (
                                    jnp.zeros((16,), jnp.float32))
                            cB = _allsum16(cB) * scale
                            state[pl.ds(l2 * stride + nd * 16, 16)] = (
                                jnp.zeros((16,), jnp.float32))
                            state[pl.ds(l2 * stride + (nd + 1) * 16, 16)] = (
                                jnp.zeros((16,), jnp.float32))
                            state[pl.ds(l2 * stride + (nd + 2) * 16, 16)] = cB
                            return _i

                        lax.fori_loop(0, nt, tinit, 0)

                        def ksuper(ks, _k):
                            kb0 = klo + ks * KCAP
                            for sb in range(KCAP // 16):
                                @pl.when(ks * KCAP + sb * 16 < kcnt)
                                def _():
                                    aa = kb0 + sb * 16
                                    aa8 = pl.multiple_of((aa // 8) * 8, 8)
                                    pltpu.sync_copy(
                                        srcs_h.at[pl.ds(aa8, 24)], sbuf)
                                    kids = sbuf[pl.ds(aa - aa8, 16)]
                                    pltpu.async_copy(
                                        pkvA_h.at[kids],
                                        kvc.at[pl.ds(sb * 16, 16)], sem2).wait()

                            def target(l2, _t2):
                                qs = [qbuf[l2, pl.ds(jj * 16, 16)]
                                      for jj in range(nd)]
                                accs = [state[pl.ds(l2 * stride + jj * 16, 16)]
                                        for jj in range(nd)]
                                den = state[pl.ds(l2 * stride + nd * 16, 16)]
                                s0 = state[pl.ds(l2 * stride + (nd + 1) * 16, 16)]
                                cB = state[pl.ds(l2 * stride + (nd + 2) * 16, 16)]

                                def kbatch(kb, c2):
                                    accs2, den2, s02 = c2
                                    accs2 = list(accs2)
                                    for l3 in range(16):
                                        kidx = ks * KCAP + kb * 16 + l3
                                        dot = jnp.zeros((16,), jnp.float32)
                                        for jj in range(nd):
                                            dot = dot + (
                                                qs[jj] * kvc[kb * 16 + l3,
                                                             pl.ds(jj * 16, 16)])
                                        s = _allsum16(dot) * scale + cB
                                        s02 = jnp.where(kidx == 0, s, s02)
                                        wgt = jnp.where(kidx < kcnt,
                                                        jnp.exp(s - s02), 0.0)
                                        den2 = den2 + wgt
                                        for jj in range(nd):
                                            accs2[jj] = accs2[jj] + wgt * kvc[
                                                kb * 16 + l3,
                                                pl.ds(d + jj * 16, 16)]
                                    return tuple(accs2), den2, s02

                                nkb = jnp.minimum(
                                    (kcnt - ks * KCAP + 15) // 16, KCAP // 16)
                                accs, den, s0 = lax.fori_loop(
                                    0, nkb, kbatch, (tuple(accs), den, s0))
                                for jj in range(nd):
                                    state[pl.ds(l2 * stride + jj * 16, 16)] = accs[jj]
                                state[pl.ds(l2 * stride + nd * 16, 16)] = den
                                state[pl.ds(l2 * stride + (nd + 1) * 16, 16)] = s0
                                return _t2

                            lax.fori_loop(0, nt, target, 0)
                            return _k

                        lax.fori_loop(0, nks, ksuper, 0)

                        def tfin(l2, _f):
                            den = state[pl.ds(l2 * stride + nd * 16, 16)]
                            rden = jnp.where(den > 0.0, 1.0 / den, 0.0)
                            has = den > 0.0
                            for jj in range(nd):
                                acc = state[pl.ds(l2 * stride + jj * 16, 16)]
                                vB = bbuf[l, pl.ds(d + jj * 16, 16)]
                                obuf[l2, pl.ds(jj * 16, 16)] = (
                                    acc * rden + jnp.where(has, vB, 0.0))
                            return _f

                        lax.fori_loop(0, nt, tfin, 0)
                        def zfill(l2, _z):
                            for jj in range(nd):
                                obuf[l2, pl.ds(jj * 16, 16)] = jnp.zeros(
                                    (16,), jnp.float32)
                            return _z
                        lax.fori_loop(nt, 16, zfill, 0)
                        pltpu.async_copy(obuf, out_h.at[pids], sem1).wait()
                        return _t

                    lax.fori_loop(0, ntb, tbatch, 0)

                return _n

            lax.fori_loop(0, 16, node, 0)
            return _g

        lax.fori_loop(0, NPW // 16, group, 0)

    return k(pqA, pqB, pkvA, pkvB, srcs_pad, dsts_pad, perm_pad, inst, outst)


def kernel(x, edge_index, Wq1, Wk1, Wv1, g1, b1, Wq2, Wk2, Wv2, g2, b2, Wq3, Wk3, Wv3):
    n_nodes = x.shape[0]
    n_edges = edge_index.shape[1]
    src = edge_index[0].astype(jnp.int32)
    dst = edge_index[1].astype(jnp.int32)

    perm_d = jnp.argsort(dst)
    perm_s = jnp.argsort(src)
    dst_sorted = dst[perm_d]
    srcs_d = src[perm_d]
    dsts_s = dst[perm_s]
    pad0 = jnp.zeros((32,), jnp.int32)
    srcs_pad = jnp.concatenate([srcs_d, pad0])
    dsts_pad = jnp.concatenate([dsts_s, pad0])
    perm_pad = jnp.concatenate([perm_s.astype(jnp.int32),
                                jnp.full((32,), n_edges, jnp.int32)])
    vr = jnp.arange(INST_LEN, dtype=jnp.int32)
    inst = jnp.searchsorted(dst_sorted, vr).astype(jnp.int32)
    src_sorted = src[perm_s]
    outst = jnp.searchsorted(src_sorted, vr).astype(jnp.int32)

    q1, kv1 = _proj1(x, Wq1, jnp.concatenate([Wk1, Wv1], axis=1))
    o1 = _seg_attn_sc(q1, kv1, srcs_pad, inst, 3)[:n_nodes]
    q2, kv2 = _ln_proj(o1, g1, b1, Wq2, jnp.concatenate([Wk2, Wv2], axis=1))
    o2 = _seg_attn_sc(q2, kv2, srcs_pad, inst, 1)[:n_nodes]
    pqA, pqB, pkvA, pkvB = _ln_proj4(
        o2, g2, b2,
        Wq3[:256], Wq3[256:],
        jnp.concatenate([Wk3[:256], Wv3[:256]], axis=1),
        jnp.concatenate([Wk3[256:], Wv3[256:]], axis=1),
    )
    out = _line_attn_sc(pqA, pqB, pkvA, pkvB, srcs_pad, dsts_pad, perm_pad,
                        inst, outst)
    return out[:n_edges]


# speedup vs baseline: 17.2972x; 1.0383x over previous
import functools
import numpy as np
import jax
import jax.numpy as jnp
from jax import lax
from jax.experimental import pallas as pl
from jax.experimental.pallas import tpu as pltpu
from jax.experimental.pallas import tpu_sc as plsc

N_NODES_C = 10000
N_EDGES_C = 160000
ROW_BLK = 200

SC_NC = 2
SC_NS = 16
SC_NW = SC_NC * SC_NS
NPW = 320
NPAD = SC_NW * NPW
ST_LEN = 336
INST_LEN = NPW * (SC_NW - 1) + ST_LEN


def _lane_gather(x, idx):
    dn = lax.GatherDimensionNumbers(offset_dims=(), collapsed_slice_dims=(0,),
                                    start_index_map=(0,))
    return lax.gather(x, idx[:, None], dn, slice_sizes=(1,),
                      mode=lax.GatherScatterMode.PROMISE_IN_BOUNDS)


def _allsum16(x):
    i = lax.iota(jnp.int32, 16)
    for k in (1, 2, 4, 8):
        x = x + _lane_gather(x, i ^ k)
    return x


def _proj_body(x_ref, wq_ref, wkv_ref, q_ref, kv_ref):
    xb = x_ref[...]
    q_ref[...] = jnp.dot(xb, wq_ref[...], preferred_element_type=jnp.float32)
    kv_ref[...] = jnp.dot(xb, wkv_ref[...], preferred_element_type=jnp.float32)


def _proj1(x, Wq, Wkv):
    n, f = x.shape
    dq, dkv = Wq.shape[1], Wkv.shape[1]
    grid = n // ROW_BLK
    return pl.pallas_call(
        _proj_body,
        grid=(grid,),
        in_specs=[
            pl.BlockSpec((ROW_BLK, f), lambda i: (i, 0)),
            pl.BlockSpec((f, dq), lambda i: (0, 0)),
            pl.BlockSpec((f, dkv), lambda i: (0, 0)),
        ],
        out_specs=[
            pl.BlockSpec((ROW_BLK, dq), lambda i: (i, 0)),
            pl.BlockSpec((ROW_BLK, dkv), lambda i: (i, 0)),
        ],
        out_shape=[
            jax.ShapeDtypeStruct((n, dq), jnp.float32),
            jax.ShapeDtypeStruct((n, dkv), jnp.float32),
        ],
    )(x, Wq, Wkv)


def _ln_proj_body(h_ref, g_ref, b_ref, wq_ref, wkv_ref, q_ref, kv_ref):
    h = h_ref[...]
    mu = jnp.mean(h, axis=-1, keepdims=True)
    var = jnp.mean(jnp.square(h - mu), axis=-1, keepdims=True)
    hn = (h - mu) * lax.rsqrt(var + 1e-5) * g_ref[...] + b_ref[...]
    hn = jnp.maximum(hn, 0.0)
    q_ref[...] = jnp.dot(hn, wq_ref[...], preferred_element_type=jnp.float32)
    kv_ref[...] = jnp.dot(hn, wkv_ref[...], preferred_element_type=jnp.float32)


def _ln_proj(h, g, b, Wq, Wkv):
    n, f = h.shape
    dq, dkv = Wq.shape[1], Wkv.shape[1]
    grid = n // ROW_BLK
    return pl.pallas_call(
        _ln_proj_body,
        grid=(grid,),
        in_specs=[
            pl.BlockSpec((ROW_BLK, f), lambda i: (i, 0)),
            pl.BlockSpec((f,), lambda i: (0,)),
            pl.BlockSpec((f,), lambda i: (0,)),
            pl.BlockSpec((f, dq), lambda i: (0, 0)),
            pl.BlockSpec((f, dkv), lambda i: (0, 0)),
        ],
        out_specs=[
            pl.BlockSpec((ROW_BLK, dq), lambda i: (i, 0)),
            pl.BlockSpec((ROW_BLK, dkv), lambda i: (i, 0)),
        ],
        out_shape=[
            jax.ShapeDtypeStruct((n, dq), jnp.float32),
            jax.ShapeDtypeStruct((n, dkv), jnp.float32),
        ],
    )(h, g, b, Wq, Wkv)


def _ln_proj4_body(h_ref, g_ref, b_ref, wa_ref, wb_ref, wc_ref, wd_ref,
                   oa_ref, ob_ref, oc_ref, od_ref):
    h = h_ref[...]
    mu = jnp.mean(h, axis=-1, keepdims=True)
    var = jnp.mean(jnp.square(h - mu), axis=-1, keepdims=True)
    hn = (h - mu) * lax.rsqrt(var + 1e-5) * g_ref[...] + b_ref[...]
    hn = jnp.maximum(hn, 0.0)
    oa_ref[...] = jnp.dot(hn, wa_ref[...], preferred_element_type=jnp.float32)
    ob_ref[...] = jnp.dot(hn, wb_ref[...], preferred_element_type=jnp.float32)
    oc_ref[...] = jnp.dot(hn, wc_ref[...], preferred_element_type=jnp.float32)
    od_ref[...] = jnp.dot(hn, wd_ref[...], preferred_element_type=jnp.float32)


def _ln_proj4(h, g, b, Wa, Wb, Wc, Wd):
    n, f = h.shape
    dims = [W.shape[1] for W in (Wa, Wb, Wc, Wd)]
    grid = n // ROW_BLK
    return pl.pallas_call(
        _ln_proj4_body,
        grid=(grid,),
        in_specs=[
            pl.BlockSpec((ROW_BLK, f), lambda i: (i, 0)),
            pl.BlockSpec((f,), lambda i: (0,)),
            pl.BlockSpec((f,), lambda i: (0,)),
        ] + [pl.BlockSpec((f, d), lambda i: (0, 0)) for d in dims],
        out_specs=[pl.BlockSpec((ROW_BLK, d), lambda i: (i, 0)) for d in dims],
        out_shape=[jax.ShapeDtypeStruct((n, d), jnp.float32) for d in dims],
    )(h, g, b, Wa, Wb, Wc, Wd)


def _seg_attn_sc(q, kv, srcs_pad, inst, heads):
    n, dq = q.shape
    dkv = kv.shape[1]
    dh = dq // heads
    ncq = dh // 16
    scale = 1.0 / np.sqrt(float(dh))
    mesh = plsc.VectorSubcoreMesh(core_axis_name="c", subcore_axis_name="s")

    @functools.partial(
        pl.kernel, mesh=mesh,
        out_type=jax.ShapeDtypeStruct((NPAD, dq), jnp.float32),
        scratch_types=[
            pltpu.VMEM((ST_LEN,), jnp.int32),
            pltpu.VMEM((24,), jnp.int32),
            pltpu.VMEM((16, dq), jnp.float32),
            pltpu.VMEM((16, dkv), jnp.float32),
            pltpu.VMEM((16, dkv), jnp.float32),
            pltpu.VMEM((16, dq), jnp.float32),
            pltpu.SemaphoreType.DMA,
            pltpu.SemaphoreType.DMA,
            pltpu.SemaphoreType.DMA,
        ],
    )
    def k(q_h, kv_h, srcs_h, inst_h, out_h, st_v, ibuf, qbuf, kvbufA, kvbufB,
          obuf, semq, semA, semB):
        w = lax.axis_index("s") * SC_NC + lax.axis_index("c")
        base = w * NPW
        pltpu.sync_copy(inst_h.at[pl.ds(pl.multiple_of(base, 8), ST_LEN)], st_v)

        def group(t, _g):
            v0 = base + t * 16
            ids = jnp.minimum(v0 + lax.iota(jnp.int32, 16), n - 1)
            pltpu.async_copy(q_h.at[ids], qbuf, semq).wait()

            def node(l, _n):
                stv = st_v[pl.ds(t * 16 + l, 16)]
                lo = stv[0]
                hi = stv[1]
                cnt = hi - lo
                for jj in range(dq // 16):
                    obuf[l, pl.ds(jj * 16, 16)] = jnp.zeros((16,), jnp.float32)
                nb = (cnt + 15) // 16

                def fire(b, buf, sem):
                    a = lo + b * 16
                    a8 = pl.multiple_of((a // 8) * 8, 8)
                    pltpu.sync_copy(srcs_h.at[pl.ds(a8, 24)], ibuf)
                    ids_e = ibuf[pl.ds(a - a8, 16)]
                    pltpu.async_copy(kv_h.at[ids_e], buf, sem)

                def drain(buf, sem):
                    pltpu.make_async_copy(kv_h.at[pl.ds(0, 16)], buf, sem).wait()

                def consume(b, buf, carry):
                    def edge(l2, carry2):
                        dens2, s0s2 = carry2
                        j = b * 16 + l2
                        new_d, new_s = [], []
                        for h in range(heads):
                            dot = jnp.zeros((16,), jnp.float32)
                            for jj in range(ncq):
                                c0 = h * dh + jj * 16
                                dot = dot + (qbuf[l, pl.ds(c0, 16)]
                                             * buf[l2, pl.ds(c0, 16)])
                            s = _allsum16(dot) * scale
                            s0h = jnp.where(j == 0, s, s0s2[h])
                            wgt = jnp.exp(s - s0h)
                            new_d.append(dens2[h] + wgt)
                            new_s.append(s0h)
                            for jj in range(ncq):
                                c0 = h * dh + jj * 16
                                obuf[l, pl.ds(c0, 16)] = (
                                    obuf[l, pl.ds(c0, 16)]
                                    + wgt * buf[l2, pl.ds(dq + c0, 16)])
                        return tuple(new_d), tuple(new_s)

                    return lax.fori_loop(0, jnp.minimum(16, cnt - b * 16),
                                         edge, carry)

                @pl.when(nb > 0)
                def _():
                    fire(0, kvbufA, semA)

                def pair(i, carry):
                    b0 = 2 * i
                    b1 = 2 * i + 1

                    @pl.when(b1 < nb)
                    def _():
                        fire(b1, kvbufB, semB)
                    drain(kvbufA, semA)
                    carry = consume(b0, kvbufA, carry)

                    @pl.when(b1 + 1 < nb)
                    def _():
                        fire(b1 + 1, kvbufA, semA)

                    @pl.when(b1 < nb)
                    def _():
                        drain(kvbufB, semB)
                    carry = consume(b1, kvbufB, carry)
                    return carry

                zero = tuple(jnp.zeros((16,), jnp.float32) for _ in range(heads))
                dens, _ = lax.fori_loop(0, (nb + 1) // 2, pair, (zero, zero))
                for h in range(heads):
                    rden = jnp.where(dens[h] > 0.0, 1.0 / dens[h], 0.0)
                    for jj in range(ncq):
                        c0 = h * dh + jj * 16
                        obuf[l, pl.ds(c0, 16)] = obuf[l, pl.ds(c0, 16)] * rden
                return _n

            lax.fori_loop(0, 16, node, 0)
            pltpu.sync_copy(obuf, out_h.at[pl.ds(pl.multiple_of(v0, 8), 16)])
            return _g

        lax.fori_loop(0, NPW // 16, group, 0)

    return k(q, kv, srcs_pad, inst)


def _line_attn_sc(pqA, pqB, pkvA, pkvB, srcs_pad, dsts_pad, perm_pad, inst, outst):
    n, d = pqA.shape
    nd = d // 16
    e_pad = perm_pad.shape[0] - 32
    scale = 1.0 / np.sqrt(float(d))
    KCAP = 64
    mesh = plsc.VectorSubcoreMesh(core_axis_name="c", subcore_axis_name="s")

    @functools.partial(
        pl.kernel, mesh=mesh,
        out_type=jax.ShapeDtypeStruct((e_pad + 16, d), jnp.float32),
        scratch_types=[
            pltpu.VMEM((ST_LEN,), jnp.int32),
            pltpu.VMEM((ST_LEN,), jnp.int32),
            pltpu.VMEM((24,), jnp.int32),
            pltpu.VMEM((24,), jnp.int32),
            pltpu.VMEM((24,), jnp.int32),
            pltpu.VMEM((16, d), jnp.float32),
            pltpu.VMEM((16, 2 * d), jnp.float32),
            pltpu.VMEM((16, d), jnp.float32),
            pltpu.VMEM((KCAP, 2 * d), jnp.float32),
            pltpu.VMEM((16, d), jnp.float32),
            pltpu.VMEM((16 * (nd + 3) * 16,), jnp.float32),
            pltpu.SemaphoreType.DMA,
            pltpu.SemaphoreType.DMA,
        ],
    )
    def k(pqA_h, pqB_h, pkvA_h, pkvB_h, srcs_h, dsts_h, perm_h, inst_h,
          outst_h, out_h, st_in, st_out, sbuf, dbuf, pbuf, abuf, bbuf, qbuf,
          kvc, obuf, state, sem1, sem2):
        w = lax.axis_index("s") * SC_NC + lax.axis_index("c")
        base = w * NPW
        pltpu.sync_copy(inst_h.at[pl.ds(pl.multiple_of(base, 8), ST_LEN)], st_in)
        pltpu.sync_copy(outst_h.at[pl.ds(pl.multiple_of(base, 8), ST_LEN)], st_out)
        stride = (nd + 3) * 16

        def group(t, _g):
            v0 = base + t * 16
            ids = jnp.minimum(v0 + lax.iota(jnp.int32, 16), n - 1)
            pltpu.async_copy(pqA_h.at[ids], abuf, sem1).wait()
            pltpu.async_copy(pkvB_h.at[ids], bbuf, sem1).wait()

            def node(l, _n):
                sti = st_in[pl.ds(t * 16 + l, 16)]
                klo = sti[0]
                kcnt = sti[1] - klo
                sto = st_out[pl.ds(t * 16 + l, 16)]
                jlo = sto[0]
                ocnt = sto[1] - jlo

                @pl.when(ocnt > 0)
                def _():
                    nks = (kcnt + KCAP - 1) // KCAP
                    ntb = (ocnt + 15) // 16

                    def tbatch(tb, _t):
                        j0 = jlo + tb * 16
                        rem = ocnt - tb * 16
                        a8 = pl.multiple_of((j0 // 8) * 8, 8)
                        pltpu.sync_copy(dsts_h.at[pl.ds(a8, 24)], dbuf)
                        tids = dbuf[pl.ds(j0 - a8, 16)]
                        pltpu.async_copy(pqB_h.at[tids], qbuf, sem1).wait()
                        pltpu.sync_copy(perm_h.at[pl.ds(a8, 24)], pbuf)
                        pids = pbuf[pl.ds(j0 - a8, 16)]
                        pids = jnp.where(lax.iota(jnp.int32, 16)
                                         < jnp.minimum(rem, 16), pids, e_pad)
                        nt = jnp.minimum(16, rem)

                        def tinit(l2, _i):
                            cB = jnp.zeros((16,), jnp.float32)
                            for jj in range(nd):
                                qv = (qbuf[l2, pl.ds(jj * 16, 16)]
                                      + abuf[l, pl.ds(jj * 16, 16)])
                                qbuf[l2, pl.ds(jj * 16, 16)] = qv
                                cB = cB + qv * bbuf[l, pl.ds(jj * 16, 16)]
                                state[pl.ds(l2 * stride + jj * 16, 16)] = (
                                    jnp.zeros((16,), jnp.float32))
                            cB = _allsum16(cB) * scale
                            state[pl.ds(l2 * stride + nd * 16, 16)] = (
                                jnp.zeros((16,), jnp.float32))
                            state[pl.ds(l2 * stride + (nd + 1) * 16, 16)] = (
                                jnp.zeros((16,), jnp.float32))
                            state[pl.ds(l2 * stride + (nd + 2) * 16, 16)] = cB
                            return _i

                        lax.fori_loop(0, nt, tinit, 0)

                        def ksuper(ks, _k):
                            kb0 = klo + ks * KCAP
                            for sb in range(KCAP // 16):
                                @pl.when(ks * KCAP + sb * 16 < kcnt)
                                def _():
                                    aa = kb0 + sb * 16
                                    aa8 = pl.multiple_of((aa // 8) * 8, 8)
                                    pltpu.sync_copy(
                                        srcs_h.at[pl.ds(aa8, 24)], sbuf)
                                    kids = sbuf[pl.ds(aa - aa8, 16)]
                                    pltpu.async_copy(
                                        pkvA_h.at[kids],
                                        kvc.at[pl.ds(sb * 16, 16)], sem2)
                            for sb in range(KCAP // 16):
                                @pl.when(ks * KCAP + sb * 16 < kcnt)
                                def _():
                                    pltpu.make_async_copy(
                                        pkvA_h.at[pl.ds(0, 16)],
                                        kvc.at[pl.ds(sb * 16, 16)], sem2).wait()

                            def target(l2, _t2):
                                qs = [qbuf[l2, pl.ds(jj * 16, 16)]
                                      for jj in range(nd)]
                                accs = [state[pl.ds(l2 * stride + jj * 16, 16)]
                                        for jj in range(nd)]
                                den = state[pl.ds(l2 * stride + nd * 16, 16)]
                                s0 = state[pl.ds(l2 * stride + (nd + 1) * 16, 16)]
                                cB = state[pl.ds(l2 * stride + (nd + 2) * 16, 16)]

                                def kbatch(kb, c2):
                                    accs2, den2, s02 = c2
                                    accs2 = list(accs2)
                                    for l3 in range(16):
                                        kidx = ks * KCAP + kb * 16 + l3
                                        dot = jnp.zeros((16,), jnp.float32)
                                        for jj in range(nd):
                                            dot = dot + (
                                                qs[jj] * kvc[kb * 16 + l3,
                                                             pl.ds(jj * 16, 16)])
                                        s = _allsum16(dot) * scale + cB
                                        s02 = jnp.where(kidx == 0, s, s02)
                                        wgt = jnp.where(kidx < kcnt,
                                                        jnp.exp(s - s02), 0.0)
                                        den2 = den2 + wgt
                                        for jj in range(nd):
                                            accs2[jj] = accs2[jj] + wgt * kvc[
                                                kb * 16 + l3,
                                                pl.ds(d + jj * 16, 16)]
                                    return tuple(accs2), den2, s02

                                nkb = jnp.minimum(
                                    (kcnt - ks * KCAP + 15) // 16, KCAP // 16)
                                accs, den, s0 = lax.fori_loop(
                                    0, nkb, kbatch, (tuple(accs), den, s0))
                                for jj in range(nd):
                                    state[pl.ds(l2 * stride + jj * 16, 16)] = accs[jj]
                                state[pl.ds(l2 * stride + nd * 16, 16)] = den
                                state[pl.ds(l2 * stride + (nd + 1) * 16, 16)] = s0
                                return _t2

                            lax.fori_loop(0, nt, target, 0)
                            return _k

                        lax.fori_loop(0, nks, ksuper, 0)

                        def tfin(l2, _f):
                            den = state[pl.ds(l2 * stride + nd * 16, 16)]
                            rden = jnp.where(den > 0.0, 1.0 / den, 0.0)
                            has = den > 0.0
                            for jj in range(nd):
                                acc = state[pl.ds(l2 * stride + jj * 16, 16)]
                                vB = bbuf[l, pl.ds(d + jj * 16, 16)]
                                obuf[l2, pl.ds(jj * 16, 16)] = (
                                    acc * rden + jnp.where(has, vB, 0.0))
                            return _f

                        lax.fori_loop(0, nt, tfin, 0)
                        def zfill(l2, _z):
                            for jj in range(nd):
                                obuf[l2, pl.ds(jj * 16, 16)] = jnp.zeros(
                                    (16,), jnp.float32)
                            return _z
                        lax.fori_loop(nt, 16, zfill, 0)
                        pltpu.async_copy(obuf, out_h.at[pids], sem1).wait()
                        return _t

                    lax.fori_loop(0, ntb, tbatch, 0)

                return _n

            lax.fori_loop(0, 16, node, 0)
            return _g

        lax.fori_loop(0, NPW // 16, group, 0)

    return k(pqA, pqB, pkvA, pkvB, srcs_pad, dsts_pad, perm_pad, inst, outst)


def kernel(x, edge_index, Wq1, Wk1, Wv1, g1, b1, Wq2, Wk2, Wv2, g2, b2, Wq3, Wk3, Wv3):
    n_nodes = x.shape[0]
    n_edges = edge_index.shape[1]
    src = edge_index[0].astype(jnp.int32)
    dst = edge_index[1].astype(jnp.int32)

    perm_d = jnp.argsort(dst)
    perm_s = jnp.argsort(src)
    dst_sorted = dst[perm_d]
    srcs_d = src[perm_d]
    dsts_s = dst[perm_s]
    pad0 = jnp.zeros((32,), jnp.int32)
    srcs_pad = jnp.concatenate([srcs_d, pad0])
    dsts_pad = jnp.concatenate([dsts_s, pad0])
    perm_pad = jnp.concatenate([perm_s.astype(jnp.int32),
                                jnp.full((32,), n_edges, jnp.int32)])
    vr = jnp.arange(INST_LEN, dtype=jnp.int32)
    inst = jnp.searchsorted(dst_sorted, vr).astype(jnp.int32)
    src_sorted = src[perm_s]
    outst = jnp.searchsorted(src_sorted, vr).astype(jnp.int32)

    q1, kv1 = _proj1(x, Wq1, jnp.concatenate([Wk1, Wv1], axis=1))
    o1 = _seg_attn_sc(q1, kv1, srcs_pad, inst, 3)[:n_nodes]
    q2, kv2 = _ln_proj(o1, g1, b1, Wq2, jnp.concatenate([Wk2, Wv2], axis=1))
    o2 = _seg_attn_sc(q2, kv2, srcs_pad, inst, 1)[:n_nodes]
    pqA, pqB, pkvA, pkvB = _ln_proj4(
        o2, g2, b2,
        Wq3[:256], Wq3[256:],
        jnp.concatenate([Wk3[:256], Wv3[:256]], axis=1),
        jnp.concatenate([Wk3[256:], Wv3[256:]], axis=1),
    )
    out = _line_attn_sc(pqA, pqB, pkvA, pkvB, srcs_pad, dsts_pad, perm_pad,
                        inst, outst)
    return out[:n_edges]


# speedup vs baseline: 17.5623x; 1.0153x over previous
import functools
import numpy as np
import jax
import jax.numpy as jnp
from jax import lax
from jax.experimental import pallas as pl
from jax.experimental.pallas import tpu as pltpu
from jax.experimental.pallas import tpu_sc as plsc

N_NODES_C = 10000
N_EDGES_C = 160000
ROW_BLK = 200

SC_NC = 2
SC_NS = 16
SC_NW = SC_NC * SC_NS
NPW = 320
NPAD = SC_NW * NPW
ST_LEN = 336
INST_LEN = NPW * (SC_NW - 1) + ST_LEN


def _lane_gather(x, idx):
    dn = lax.GatherDimensionNumbers(offset_dims=(), collapsed_slice_dims=(0,),
                                    start_index_map=(0,))
    return lax.gather(x, idx[:, None], dn, slice_sizes=(1,),
                      mode=lax.GatherScatterMode.PROMISE_IN_BOUNDS)


def _allsum16(x):
    i = lax.iota(jnp.int32, 16)
    for k in (1, 2, 4, 8):
        x = x + _lane_gather(x, i ^ k)
    return x


def _proj_body(x_ref, wq_ref, wkv_ref, q_ref, kv_ref):
    xb = x_ref[...]
    q_ref[...] = jnp.dot(xb, wq_ref[...], preferred_element_type=jnp.float32)
    kv_ref[...] = jnp.dot(xb, wkv_ref[...], preferred_element_type=jnp.float32)


def _proj1(x, Wq, Wkv):
    n, f = x.shape
    dq, dkv = Wq.shape[1], Wkv.shape[1]
    grid = n // ROW_BLK
    return pl.pallas_call(
        _proj_body,
        grid=(grid,),
        in_specs=[
            pl.BlockSpec((ROW_BLK, f), lambda i: (i, 0)),
            pl.BlockSpec((f, dq), lambda i: (0, 0)),
            pl.BlockSpec((f, dkv), lambda i: (0, 0)),
        ],
        out_specs=[
            pl.BlockSpec((ROW_BLK, dq), lambda i: (i, 0)),
            pl.BlockSpec((ROW_BLK, dkv), lambda i: (i, 0)),
        ],
        out_shape=[
            jax.ShapeDtypeStruct((n, dq), jnp.float32),
            jax.ShapeDtypeStruct((n, dkv), jnp.float32),
        ],
    )(x, Wq, Wkv)


def _ln_proj_body(h_ref, g_ref, b_ref, wq_ref, wkv_ref, q_ref, kv_ref):
    h = h_ref[...]
    mu = jnp.mean(h, axis=-1, keepdims=True)
    var = jnp.mean(jnp.square(h - mu), axis=-1, keepdims=True)
    hn = (h - mu) * lax.rsqrt(var + 1e-5) * g_ref[...] + b_ref[...]
    hn = jnp.maximum(hn, 0.0)
    q_ref[...] = jnp.dot(hn, wq_ref[...], preferred_element_type=jnp.float32)
    kv_ref[...] = jnp.dot(hn, wkv_ref[...], preferred_element_type=jnp.float32)


def _ln_proj(h, g, b, Wq, Wkv):
    n, f = h.shape
    dq, dkv = Wq.shape[1], Wkv.shape[1]
    grid = n // ROW_BLK
    return pl.pallas_call(
        _ln_proj_body,
        grid=(grid,),
        in_specs=[
            pl.BlockSpec((ROW_BLK, f), lambda i: (i, 0)),
            pl.BlockSpec((f,), lambda i: (0,)),
            pl.BlockSpec((f,), lambda i: (0,)),
            pl.BlockSpec((f, dq), lambda i: (0, 0)),
            pl.BlockSpec((f, dkv), lambda i: (0, 0)),
        ],
        out_specs=[
            pl.BlockSpec((ROW_BLK, dq), lambda i: (i, 0)),
            pl.BlockSpec((ROW_BLK, dkv), lambda i: (i, 0)),
        ],
        out_shape=[
            jax.ShapeDtypeStruct((n, dq), jnp.float32),
            jax.ShapeDtypeStruct((n, dkv), jnp.float32),
        ],
    )(h, g, b, Wq, Wkv)


def _ln_proj4_body(h_ref, g_ref, b_ref, wa_ref, wb_ref, wc_ref, wd_ref,
                   oa_ref, ob_ref, oc_ref, od_ref):
    h = h_ref[...]
    mu = jnp.mean(h, axis=-1, keepdims=True)
    var = jnp.mean(jnp.square(h - mu), axis=-1, keepdims=True)
    hn = (h - mu) * lax.rsqrt(var + 1e-5) * g_ref[...] + b_ref[...]
    hn = jnp.maximum(hn, 0.0)
    oa_ref[...] = jnp.dot(hn, wa_ref[...], preferred_element_type=jnp.float32)
    ob_ref[...] = jnp.dot(hn, wb_ref[...], preferred_element_type=jnp.float32)
    oc_ref[...] = jnp.dot(hn, wc_ref[...], preferred_element_type=jnp.float32)
    od_ref[...] = jnp.dot(hn, wd_ref[...], preferred_element_type=jnp.float32)


def _ln_proj4(h, g, b, Wa, Wb, Wc, Wd):
    n, f = h.shape
    dims = [W.shape[1] for W in (Wa, Wb, Wc, Wd)]
    grid = n // ROW_BLK
    return pl.pallas_call(
        _ln_proj4_body,
        grid=(grid,),
        in_specs=[
            pl.BlockSpec((ROW_BLK, f), lambda i: (i, 0)),
            pl.BlockSpec((f,), lambda i: (0,)),
            pl.BlockSpec((f,), lambda i: (0,)),
        ] + [pl.BlockSpec((f, d), lambda i: (0, 0)) for d in dims],
        out_specs=[pl.BlockSpec((ROW_BLK, d), lambda i: (i, 0)) for d in dims],
        out_shape=[jax.ShapeDtypeStruct((n, d), jnp.float32) for d in dims],
    )(h, g, b, Wa, Wb, Wc, Wd)


def _seg_attn_sc(q, kv, srcs_pad, inst, heads):
    n, dq = q.shape
    dkv = kv.shape[1]
    dh = dq // heads
    ncq = dh // 16
    scale = 1.0 / np.sqrt(float(dh))
    mesh = plsc.VectorSubcoreMesh(core_axis_name="c", subcore_axis_name="s")

    @functools.partial(
        pl.kernel, mesh=mesh,
        out_type=jax.ShapeDtypeStruct((NPAD, dq), jnp.float32),
        scratch_types=[
            pltpu.VMEM((ST_LEN,), jnp.int32),
            pltpu.VMEM((24,), jnp.int32),
            pltpu.VMEM((16, dq), jnp.float32),
            pltpu.VMEM((16, dkv), jnp.float32),
            pltpu.VMEM((16, dkv), jnp.float32),
            pltpu.VMEM((16, dq), jnp.float32),
            pltpu.SemaphoreType.DMA,
            pltpu.SemaphoreType.DMA,
            pltpu.SemaphoreType.DMA,
        ],
    )
    def k(q_h, kv_h, srcs_h, inst_h, out_h, st_v, ibuf, qbuf, kvbufA, kvbufB,
          obuf, semq, semA, semB):
        w = lax.axis_index("s") * SC_NC + lax.axis_index("c")
        base = w * NPW
        pltpu.sync_copy(inst_h.at[pl.ds(pl.multiple_of(base, 8), ST_LEN)], st_v)

        def group(t, _g):
            v0 = base + t * 16
            ids = jnp.minimum(v0 + lax.iota(jnp.int32, 16), n - 1)
            pltpu.async_copy(q_h.at[ids], qbuf, semq).wait()

            def node(l, _n):
                stv = st_v[pl.ds(t * 16 + l, 16)]
                lo = stv[0]
                hi = stv[1]
                cnt = hi - lo
                for jj in range(dq // 16):
                    obuf[l, pl.ds(jj * 16, 16)] = jnp.zeros((16,), jnp.float32)
                nb = (cnt + 15) // 16

                def fire(b, buf, sem):
                    a = lo + b * 16
                    a8 = pl.multiple_of((a // 8) * 8, 8)
                    pltpu.sync_copy(srcs_h.at[pl.ds(a8, 24)], ibuf)
                    ids_e = ibuf[pl.ds(a - a8, 16)]
                    pltpu.async_copy(kv_h.at[ids_e], buf, sem)

                def drain(buf, sem):
                    pltpu.make_async_copy(kv_h.at[pl.ds(0, 16)], buf, sem).wait()

                def consume(b, buf, carry):
                    def edge(l2, carry2):
                        dens2, s0s2 = carry2
                        j = b * 16 + l2
                        new_d, new_s = [], []
                        for h in range(heads):
                            parts = [jnp.zeros((16,), jnp.float32)
                                     for _ in range(4)]
                            for jj in range(ncq):
                                c0 = h * dh + jj * 16
                                parts[jj % 4] = parts[jj % 4] + (
                                    qbuf[l, pl.ds(c0, 16)]
                                    * buf[l2, pl.ds(c0, 16)])
                            dot = (parts[0] + parts[1]) + (parts[2] + parts[3])
                            s = _allsum16(dot) * scale
                            s0h = jnp.where(j == 0, s, s0s2[h])
                            wgt = jnp.exp(s - s0h)
                            new_d.append(dens2[h] + wgt)
                            new_s.append(s0h)
                            for jj in range(ncq):
                                c0 = h * dh + jj * 16
                                obuf[l, pl.ds(c0, 16)] = (
                                    obuf[l, pl.ds(c0, 16)]
                                    + wgt * buf[l2, pl.ds(dq + c0, 16)])
                        return tuple(new_d), tuple(new_s)

                    return lax.fori_loop(0, jnp.minimum(16, cnt - b * 16),
                                         edge, carry)

                @pl.when(nb > 0)
                def _():
                    fire(0, kvbufA, semA)

                def pair(i, carry):
                    b0 = 2 * i
                    b1 = 2 * i + 1

                    @pl.when(b1 < nb)
                    def _():
                        fire(b1, kvbufB, semB)
                    drain(kvbufA, semA)
                    carry = consume(b0, kvbufA, carry)

                    @pl.when(b1 + 1 < nb)
                    def _():
                        fire(b1 + 1, kvbufA, semA)

                    @pl.when(b1 < nb)
                    def _():
                        drain(kvbufB, semB)
                    carry = consume(b1, kvbufB, carry)
                    return carry

                zero = tuple(jnp.zeros((16,), jnp.float32) for _ in range(heads))
                dens, _ = lax.fori_loop(0, (nb + 1) // 2, pair, (zero, zero))
                for h in range(heads):
                    rden = jnp.where(dens[h] > 0.0, 1.0 / dens[h], 0.0)
                    for jj in range(ncq):
                        c0 = h * dh + jj * 16
                        obuf[l, pl.ds(c0, 16)] = obuf[l, pl.ds(c0, 16)] * rden
                return _n

            lax.fori_loop(0, 16, node, 0)
            pltpu.sync_copy(obuf, out_h.at[pl.ds(pl.multiple_of(v0, 8), 16)])
            return _g

        lax.fori_loop(0, NPW // 16, group, 0)

    return k(q, kv, srcs_pad, inst)


def _line_attn_sc(pqA, pqB, pkvA, pkvB, srcs_pad, dsts_pad, perm_pad, inst, outst):
    n, d = pqA.shape
    nd = d // 16
    e_pad = perm_pad.shape[0] - 32
    scale = 1.0 / np.sqrt(float(d))
    KCAP = 64
    mesh = plsc.VectorSubcoreMesh(core_axis_name="c", subcore_axis_name="s")

    @functools.partial(
        pl.kernel, mesh=mesh,
        out_type=jax.ShapeDtypeStruct((e_pad + 16, d), jnp.float32),
        scratch_types=[
            pltpu.VMEM((ST_LEN,), jnp.int32),
            pltpu.VMEM((ST_LEN,), jnp.int32),
            pltpu.VMEM((24,), jnp.int32),
            pltpu.VMEM((24,), jnp.int32),
            pltpu.VMEM((24,), jnp.int32),
            pltpu.VMEM((16, d), jnp.float32),
            pltpu.VMEM((16, 2 * d), jnp.float32),
            pltpu.VMEM((16, d), jnp.float32),
            pltpu.VMEM((KCAP, 2 * d), jnp.float32),
            pltpu.VMEM((16, d), jnp.float32),
            pltpu.VMEM((16 * (nd + 3) * 16,), jnp.float32),
            pltpu.SemaphoreType.DMA,
            pltpu.SemaphoreType.DMA,
        ],
    )
    def k(pqA_h, pqB_h, pkvA_h, pkvB_h, srcs_h, dsts_h, perm_h, inst_h,
          outst_h, out_h, st_in, st_out, sbuf, dbuf, pbuf, abuf, bbuf, qbuf,
          kvc, obuf, state, sem1, sem2):
        w = lax.axis_index("s") * SC_NC + lax.axis_index("c")
        base = w * NPW
        pltpu.sync_copy(inst_h.at[pl.ds(pl.multiple_of(base, 8), ST_LEN)], st_in)
        pltpu.sync_copy(outst_h.at[pl.ds(pl.multiple_of(base, 8), ST_LEN)], st_out)
        stride = (nd + 3) * 16

        def group(t, _g):
            v0 = base + t * 16
            ids = jnp.minimum(v0 + lax.iota(jnp.int32, 16), n - 1)
            pltpu.async_copy(pqA_h.at[ids], abuf, sem1).wait()
            pltpu.async_copy(pkvB_h.at[ids], bbuf, sem1).wait()

            def node(l, _n):
                sti = st_in[pl.ds(t * 16 + l, 16)]
                klo = sti[0]
                kcnt = sti[1] - klo
                sto = st_out[pl.ds(t * 16 + l, 16)]
                jlo = sto[0]
                ocnt = sto[1] - jlo

                @pl.when(ocnt > 0)
                def _():
                    nks = (kcnt + KCAP - 1) // KCAP
                    ntb = (ocnt + 15) // 16

                    def tbatch(tb, _t):
                        j0 = jlo + tb * 16
                        rem = ocnt - tb * 16
                        a8 = pl.multiple_of((j0 // 8) * 8, 8)
                        pltpu.sync_copy(dsts_h.at[pl.ds(a8, 24)], dbuf)
                        tids = dbuf[pl.ds(j0 - a8, 16)]
                        pltpu.async_copy(pqB_h.at[tids], qbuf, sem1).wait()
                        pltpu.sync_copy(perm_h.at[pl.ds(a8, 24)], pbuf)
                        pids = pbuf[pl.ds(j0 - a8, 16)]
                        pids = jnp.where(lax.iota(jnp.int32, 16)
                                         < jnp.minimum(rem, 16), pids, e_pad)
                        nt = jnp.minimum(16, rem)

                        def tinit(l2, _i):
                            cB = jnp.zeros((16,), jnp.float32)
                            for jj in range(nd):
                                qv = (qbuf[l2, pl.ds(jj * 16, 16)]
                                      + abuf[l, pl.ds(jj * 16, 16)])
                                qbuf[l2, pl.ds(jj * 16, 16)] = qv
                                cB = cB + qv * bbuf[l, pl.ds(jj * 16, 16)]
                                state[pl.ds(l2 * stride + jj * 16, 16)] = (
                                    jnp.zeros((16,), jnp.float32))
                            cB = _allsum16(cB) * scale
                            state[pl.ds(l2 * stride + nd * 16, 16)] = (
                                jnp.zeros((16,), jnp.float32))
                            state[pl.ds(l2 * stride + (nd + 1) * 16, 16)] = (
                                jnp.zeros((16,), jnp.float32))
                            state[pl.ds(l2 * stride + (nd + 2) * 16, 16)] = cB
                            return _i

                        lax.fori_loop(0, nt, tinit, 0)

                        def ksuper(ks, _k):
                            kb0 = klo + ks * KCAP
                            for sb in range(KCAP // 16):
                                @pl.when(ks * KCAP + sb * 16 < kcnt)
                                def _():
                                    aa = kb0 + sb * 16
                                    aa8 = pl.multiple_of((aa // 8) * 8, 8)
                                    pltpu.sync_copy(
                                        srcs_h.at[pl.ds(aa8, 24)], sbuf)
                                    kids = sbuf[pl.ds(aa - aa8, 16)]
                                    pltpu.async_copy(
                                        pkvA_h.at[kids],
                                        kvc.at[pl.ds(sb * 16, 16)], sem2)
                            for sb in range(KCAP // 16):
                                @pl.when(ks * KCAP + sb * 16 < kcnt)
                                def _():
                                    pltpu.make_async_copy(
                                        pkvA_h.at[pl.ds(0, 16)],
                                        kvc.at[pl.ds(sb * 16, 16)], sem2).wait()

                            def target(l2, _t2):
                                qs = [qbuf[l2, pl.ds(jj * 16, 16)]
                                      for jj in range(nd)]
                                accs = [state[pl.ds(l2 * stride + jj * 16, 16)]
                                        for jj in range(nd)]
                                den = state[pl.ds(l2 * stride + nd * 16, 16)]
                                s0 = state[pl.ds(l2 * stride + (nd + 1) * 16, 16)]
                                cB = state[pl.ds(l2 * stride + (nd + 2) * 16, 16)]

                                def kbatch(kb, c2):
                                    accs2, den2, s02 = c2
                                    accs2 = list(accs2)
                                    for l3 in range(16):
                                        kidx = ks * KCAP + kb * 16 + l3
                                        parts = [jnp.zeros((16,), jnp.float32)
                                                 for _ in range(4)]
                                        for jj in range(nd):
                                            parts[jj % 4] = parts[jj % 4] + (
                                                qs[jj] * kvc[kb * 16 + l3,
                                                             pl.ds(jj * 16, 16)])
                                        dot = ((parts[0] + parts[1])
                                               + (parts[2] + parts[3]))
                                        s = _allsum16(dot) * scale + cB
                                        s02 = jnp.where(kidx == 0, s, s02)
                                        wgt = jnp.where(kidx < kcnt,
                                                        jnp.exp(s - s02), 0.0)
                                        den2 = den2 + wgt
                                        for jj in range(nd):
                                            accs2[jj] = accs2[jj] + wgt * kvc[
                                                kb * 16 + l3,
                                                pl.ds(d + jj * 16, 16)]
                                    return tuple(accs2), den2, s02

                                nkb = jnp.minimum(
                                    (kcnt - ks * KCAP + 15) // 16, KCAP // 16)
                                accs, den, s0 = lax.fori_loop(
                                    0, nkb, kbatch, (tuple(accs), den, s0))
                                for jj in range(nd):
                                    state[pl.ds(l2 * stride + jj * 16, 16)] = accs[jj]
                                state[pl.ds(l2 * stride + nd * 16, 16)] = den
                                state[pl.ds(l2 * stride + (nd + 1) * 16, 16)] = s0
                                return _t2

                            lax.fori_loop(0, nt, target, 0)
                            return _k

                        lax.fori_loop(0, nks, ksuper, 0)

                        def tfin(l2, _f):
                            den = state[pl.ds(l2 * stride + nd * 16, 16)]
                            rden = jnp.where(den > 0.0, 1.0 / den, 0.0)
                            has = den > 0.0
                            for jj in range(nd):
                                acc = state[pl.ds(l2 * stride + jj * 16, 16)]
                                vB = bbuf[l, pl.ds(d + jj * 16, 16)]
                                obuf[l2, pl.ds(jj * 16, 16)] = (
                                    acc * rden + jnp.where(has, vB, 0.0))
                            return _f

                        lax.fori_loop(0, nt, tfin, 0)
                        def zfill(l2, _z):
                            for jj in range(nd):
                                obuf[l2, pl.ds(jj * 16, 16)] = jnp.zeros(
                                    (16,), jnp.float32)
                            return _z
                        lax.fori_loop(nt, 16, zfill, 0)
                        pltpu.async_copy(obuf, out_h.at[pids], sem1).wait()
                        return _t

                    lax.fori_loop(0, ntb, tbatch, 0)

                return _n

            lax.fori_loop(0, 16, node, 0)
            return _g

        lax.fori_loop(0, NPW // 16, group, 0)

    return k(pqA, pqB, pkvA, pkvB, srcs_pad, dsts_pad, perm_pad, inst, outst)


def kernel(x, edge_index, Wq1, Wk1, Wv1, g1, b1, Wq2, Wk2, Wv2, g2, b2, Wq3, Wk3, Wv3):
    n_nodes = x.shape[0]
    n_edges = edge_index.shape[1]
    src = edge_index[0].astype(jnp.int32)
    dst = edge_index[1].astype(jnp.int32)

    perm_d = jnp.argsort(dst)
    perm_s = jnp.argsort(src)
    dst_sorted = dst[perm_d]
    srcs_d = src[perm_d]
    dsts_s = dst[perm_s]
    pad0 = jnp.zeros((32,), jnp.int32)
    srcs_pad = jnp.concatenate([srcs_d, pad0])
    dsts_pad = jnp.concatenate([dsts_s, pad0])
    perm_pad = jnp.concatenate([perm_s.astype(jnp.int32),
                                jnp.full((32,), n_edges, jnp.int32)])
    vr = jnp.arange(INST_LEN, dtype=jnp.int32)
    inst = jnp.searchsorted(dst_sorted, vr).astype(jnp.int32)
    src_sorted = src[perm_s]
    outst = jnp.searchsorted(src_sorted, vr).astype(jnp.int32)

    q1, kv1 = _proj1(x, Wq1, jnp.concatenate([Wk1, Wv1], axis=1))
    o1 = _seg_attn_sc(q1, kv1, srcs_pad, inst, 3)[:n_nodes]
    q2, kv2 = _ln_proj(o1, g1, b1, Wq2, jnp.concatenate([Wk2, Wv2], axis=1))
    o2 = _seg_attn_sc(q2, kv2, srcs_pad, inst, 1)[:n_nodes]
    pqA, pqB, pkvA, pkvB = _ln_proj4(
        o2, g2, b2,
        Wq3[:256], Wq3[256:],
        jnp.concatenate([Wk3[:256], Wv3[:256]], axis=1),
        jnp.concatenate([Wk3[256:], Wv3[256:]], axis=1),
    )
    out = _line_attn_sc(pqA, pqB, pkvA, pkvB, srcs_pad, dsts_pad, perm_pad,
                        inst, outst)
    return out[:n_edges]


# speedup vs baseline: 23.3921x; 1.3320x over previous
import functools
import numpy as np
import jax
import jax.numpy as jnp
from jax import lax
from jax.experimental import pallas as pl
from jax.experimental.pallas import tpu as pltpu
from jax.experimental.pallas import tpu_sc as plsc

N_NODES_C = 10000
N_EDGES_C = 160000
ROW_BLK = 200

SC_NC = 2
SC_NS = 16
SC_NW = SC_NC * SC_NS
NPW = 320
NPAD = SC_NW * NPW
ST_LEN = 336
INST_LEN = NPW * (SC_NW - 1) + ST_LEN


def _lane_gather(x, idx):
    dn = lax.GatherDimensionNumbers(offset_dims=(), collapsed_slice_dims=(0,),
                                    start_index_map=(0,))
    return lax.gather(x, idx[:, None], dn, slice_sizes=(1,),
                      mode=lax.GatherScatterMode.PROMISE_IN_BOUNDS)


def _allsum16(x):
    i = lax.iota(jnp.int32, 16)
    for k in (1, 2, 4, 8):
        x = x + _lane_gather(x, i ^ k)
    return x


def _proj_body(x_ref, wq_ref, wkv_ref, q_ref, kv_ref):
    xb = x_ref[...]
    q_ref[...] = jnp.dot(xb, wq_ref[...], preferred_element_type=jnp.float32)
    kv_ref[...] = jnp.dot(xb, wkv_ref[...], preferred_element_type=jnp.float32)


def _proj1(x, Wq, Wkv):
    n, f = x.shape
    dq, dkv = Wq.shape[1], Wkv.shape[1]
    grid = n // ROW_BLK
    return pl.pallas_call(
        _proj_body,
        grid=(grid,),
        in_specs=[
            pl.BlockSpec((ROW_BLK, f), lambda i: (i, 0)),
            pl.BlockSpec((f, dq), lambda i: (0, 0)),
            pl.BlockSpec((f, dkv), lambda i: (0, 0)),
        ],
        out_specs=[
            pl.BlockSpec((ROW_BLK, dq), lambda i: (i, 0)),
            pl.BlockSpec((ROW_BLK, dkv), lambda i: (i, 0)),
        ],
        out_shape=[
            jax.ShapeDtypeStruct((n, dq), jnp.float32),
            jax.ShapeDtypeStruct((n, dkv), jnp.float32),
        ],
    )(x, Wq, Wkv)


def _ln_proj_body(h_ref, g_ref, b_ref, wq_ref, wkv_ref, q_ref, kv_ref):
    h = h_ref[...]
    mu = jnp.mean(h, axis=-1, keepdims=True)
    var = jnp.mean(jnp.square(h - mu), axis=-1, keepdims=True)
    hn = (h - mu) * lax.rsqrt(var + 1e-5) * g_ref[...] + b_ref[...]
    hn = jnp.maximum(hn, 0.0)
    q_ref[...] = jnp.dot(hn, wq_ref[...], preferred_element_type=jnp.float32)
    kv_ref[...] = jnp.dot(hn, wkv_ref[...], preferred_element_type=jnp.float32)


def _ln_proj(h, g, b, Wq, Wkv):
    n, f = h.shape
    dq, dkv = Wq.shape[1], Wkv.shape[1]
    grid = n // ROW_BLK
    return pl.pallas_call(
        _ln_proj_body,
        grid=(grid,),
        in_specs=[
            pl.BlockSpec((ROW_BLK, f), lambda i: (i, 0)),
            pl.BlockSpec((f,), lambda i: (0,)),
            pl.BlockSpec((f,), lambda i: (0,)),
            pl.BlockSpec((f, dq), lambda i: (0, 0)),
            pl.BlockSpec((f, dkv), lambda i: (0, 0)),
        ],
        out_specs=[
            pl.BlockSpec((ROW_BLK, dq), lambda i: (i, 0)),
            pl.BlockSpec((ROW_BLK, dkv), lambda i: (i, 0)),
        ],
        out_shape=[
            jax.ShapeDtypeStruct((n, dq), jnp.float32),
            jax.ShapeDtypeStruct((n, dkv), jnp.float32),
        ],
    )(h, g, b, Wq, Wkv)


def _ln_proj4_body(h_ref, g_ref, b_ref, wa_ref, wb_ref, wc_ref, wd_ref,
                   oa_ref, ob_ref, oc_ref, od_ref):
    h = h_ref[...]
    mu = jnp.mean(h, axis=-1, keepdims=True)
    var = jnp.mean(jnp.square(h - mu), axis=-1, keepdims=True)
    hn = (h - mu) * lax.rsqrt(var + 1e-5) * g_ref[...] + b_ref[...]
    hn = jnp.maximum(hn, 0.0)
    oa_ref[...] = jnp.dot(hn, wa_ref[...], preferred_element_type=jnp.float32)
    ob_ref[...] = jnp.dot(hn, wb_ref[...], preferred_element_type=jnp.float32)
    oc_ref[...] = jnp.dot(hn, wc_ref[...], preferred_element_type=jnp.float32)
    od_ref[...] = jnp.dot(hn, wd_ref[...], preferred_element_type=jnp.float32)


def _ln_proj4(h, g, b, Wa, Wb, Wc, Wd):
    n, f = h.shape
    dims = [W.shape[1] for W in (Wa, Wb, Wc, Wd)]
    grid = n // ROW_BLK
    return pl.pallas_call(
        _ln_proj4_body,
        grid=(grid,),
        in_specs=[
            pl.BlockSpec((ROW_BLK, f), lambda i: (i, 0)),
            pl.BlockSpec((f,), lambda i: (0,)),
            pl.BlockSpec((f,), lambda i: (0,)),
        ] + [pl.BlockSpec((f, d), lambda i: (0, 0)) for d in dims],
        out_specs=[pl.BlockSpec((ROW_BLK, d), lambda i: (i, 0)) for d in dims],
        out_shape=[jax.ShapeDtypeStruct((n, d), jnp.float32) for d in dims],
    )(h, g, b, Wa, Wb, Wc, Wd)


def _seg_attn_sc(q, kv, srcs_pad, inst, heads):
    n, dq = q.shape
    dkv = kv.shape[1]
    dh = dq // heads
    ncq = dh // 16
    scale = 1.0 / np.sqrt(float(dh))
    mesh = plsc.VectorSubcoreMesh(core_axis_name="c", subcore_axis_name="s")

    @functools.partial(
        pl.kernel, mesh=mesh,
        out_type=jax.ShapeDtypeStruct((NPAD, dq), jnp.float32),
        scratch_types=[
            pltpu.VMEM((ST_LEN,), jnp.int32),
            pltpu.VMEM((24,), jnp.int32),
            pltpu.VMEM((16, dq), jnp.float32),
            pltpu.VMEM((16, dkv), jnp.float32),
            pltpu.VMEM((16, dkv), jnp.float32),
            pltpu.VMEM((16, dq), jnp.float32),
            pltpu.SemaphoreType.DMA,
            pltpu.SemaphoreType.DMA,
            pltpu.SemaphoreType.DMA,
        ],
    )
    def k(q_h, kv_h, srcs_h, inst_h, out_h, st_v, ibuf, qbuf, kvbufA, kvbufB,
          obuf, semq, semA, semB):
        w = lax.axis_index("s") * SC_NC + lax.axis_index("c")
        base = w * NPW
        pltpu.sync_copy(inst_h.at[pl.ds(pl.multiple_of(base, 8), ST_LEN)], st_v)

        def group(t, _g):
            v0 = base + t * 16
            ids = jnp.minimum(v0 + lax.iota(jnp.int32, 16), n - 1)
            pltpu.async_copy(q_h.at[ids], qbuf, semq).wait()

            def node(l, _n):
                stv = st_v[pl.ds(t * 16 + l, 16)]
                lo = stv[0]
                hi = stv[1]
                cnt = hi - lo
                for jj in range(dq // 16):
                    obuf[l, pl.ds(jj * 16, 16)] = jnp.zeros((16,), jnp.float32)
                nb = (cnt + 15) // 16

                def fire(b, buf, sem):
                    a = lo + b * 16
                    a8 = pl.multiple_of((a // 8) * 8, 8)
                    pltpu.sync_copy(srcs_h.at[pl.ds(a8, 24)], ibuf)
                    ids_e = ibuf[pl.ds(a - a8, 16)]
                    pltpu.async_copy(kv_h.at[ids_e], buf, sem)

                def drain(buf, sem):
                    pltpu.make_async_copy(kv_h.at[pl.ds(0, 16)], buf, sem).wait()

                def consume(b, buf, carry):
                    def edge(l2, carry2):
                        dens2, s0s2 = carry2
                        j = b * 16 + l2
                        new_d, new_s = [], []
                        for h in range(heads):
                            parts = [jnp.zeros((16,), jnp.float32)
                                     for _ in range(4)]
                            for jj in range(ncq):
                                c0 = h * dh + jj * 16
                                parts[jj % 4] = parts[jj % 4] + (
                                    qbuf[l, pl.ds(c0, 16)]
                                    * buf[l2, pl.ds(c0, 16)])
                            dot = (parts[0] + parts[1]) + (parts[2] + parts[3])
                            s = _allsum16(dot) * scale
                            s0h = jnp.where(j == 0, s, s0s2[h])
                            wgt = jnp.exp(s - s0h)
                            new_d.append(dens2[h] + wgt)
                            new_s.append(s0h)
                            for jj in range(ncq):
                                c0 = h * dh + jj * 16
                                obuf[l, pl.ds(c0, 16)] = (
                                    obuf[l, pl.ds(c0, 16)]
                                    + wgt * buf[l2, pl.ds(dq + c0, 16)])
                        return tuple(new_d), tuple(new_s)

                    return lax.fori_loop(0, jnp.minimum(16, cnt - b * 16),
                                         edge, carry)

                @pl.when(nb > 0)
                def _():
                    fire(0, kvbufA, semA)

                def pair(i, carry):
                    b0 = 2 * i
                    b1 = 2 * i + 1

                    @pl.when(b1 < nb)
                    def _():
                        fire(b1, kvbufB, semB)
                    drain(kvbufA, semA)
                    carry = consume(b0, kvbufA, carry)

                    @pl.when(b1 + 1 < nb)
                    def _():
                        fire(b1 + 1, kvbufA, semA)

                    @pl.when(b1 < nb)
                    def _():
                        drain(kvbufB, semB)
                    carry = consume(b1, kvbufB, carry)
                    return carry

                zero = tuple(jnp.zeros((16,), jnp.float32) for _ in range(heads))
                dens, _ = lax.fori_loop(0, (nb + 1) // 2, pair, (zero, zero))
                for h in range(heads):
                    rden = jnp.where(dens[h] > 0.0, 1.0 / dens[h], 0.0)
                    for jj in range(ncq):
                        c0 = h * dh + jj * 16
                        obuf[l, pl.ds(c0, 16)] = obuf[l, pl.ds(c0, 16)] * rden
                return _n

            lax.fori_loop(0, 16, node, 0)
            pltpu.sync_copy(obuf, out_h.at[pl.ds(pl.multiple_of(v0, 8), 16)])
            return _g

        lax.fori_loop(0, NPW // 16, group, 0)

    return k(q, kv, srcs_pad, inst)


def _line_attn_sc(pqA, pqB, pkvA, pkvB, srcs_pad, dsts_pad, perm_pad, inst, outst):
    n, d = pqA.shape
    nd = d // 16
    e_pad = perm_pad.shape[0] - 32
    scale = 1.0 / np.sqrt(float(d))
    KCAP = 64
    mesh = plsc.VectorSubcoreMesh(core_axis_name="c", subcore_axis_name="s")

    @functools.partial(
        pl.kernel, mesh=mesh,
        out_type=jax.ShapeDtypeStruct((e_pad + 16, d), jnp.float32),
        scratch_types=[
            pltpu.VMEM((ST_LEN,), jnp.int32),
            pltpu.VMEM((ST_LEN,), jnp.int32),
            pltpu.VMEM((24,), jnp.int32),
            pltpu.VMEM((24,), jnp.int32),
            pltpu.VMEM((24,), jnp.int32),
            pltpu.VMEM((16, d), jnp.float32),
            pltpu.VMEM((16, 2 * d), jnp.float32),
            pltpu.VMEM((16, d), jnp.float32),
            pltpu.VMEM((KCAP, 2 * d), jnp.float32),
            pltpu.VMEM((16, d), jnp.float32),
            pltpu.VMEM((16 * (nd + 3) * 16,), jnp.float32),
            pltpu.SemaphoreType.DMA,
            pltpu.SemaphoreType.DMA,
        ],
    )
    def k(pqA_h, pqB_h, pkvA_h, pkvB_h, srcs_h, dsts_h, perm_h, inst_h,
          outst_h, out_h, st_in, st_out, sbuf, dbuf, pbuf, abuf, bbuf, qbuf,
          kvc, obuf, state, sem1, sem2):
        w = lax.axis_index("s") * SC_NC + lax.axis_index("c")
        base = w * NPW
        pltpu.sync_copy(inst_h.at[pl.ds(pl.multiple_of(base, 8), ST_LEN)], st_in)
        pltpu.sync_copy(outst_h.at[pl.ds(pl.multiple_of(base, 8), ST_LEN)], st_out)
        stride = (nd + 3) * 16

        def group(t, _g):
            v0 = base + t * 16
            ids = jnp.minimum(v0 + lax.iota(jnp.int32, 16), n - 1)
            pltpu.async_copy(pqA_h.at[ids], abuf, sem1).wait()
            pltpu.async_copy(pkvB_h.at[ids], bbuf, sem1).wait()

            def node(l, _n):
                sti = st_in[pl.ds(t * 16 + l, 16)]
                klo = sti[0]
                kcnt = sti[1] - klo
                sto = st_out[pl.ds(t * 16 + l, 16)]
                jlo = sto[0]
                ocnt = sto[1] - jlo

                @pl.when(ocnt > 0)
                def _():
                    nks = (kcnt + KCAP - 1) // KCAP
                    ntb = (ocnt + 15) // 16

                    def tbatch(tb, _t):
                        j0 = jlo + tb * 16
                        rem = ocnt - tb * 16
                        a8 = pl.multiple_of((j0 // 8) * 8, 8)
                        pltpu.sync_copy(dsts_h.at[pl.ds(a8, 24)], dbuf)
                        tids = dbuf[pl.ds(j0 - a8, 16)]
                        pltpu.async_copy(pqB_h.at[tids], qbuf, sem1).wait()
                        pltpu.sync_copy(perm_h.at[pl.ds(a8, 24)], pbuf)
                        pids = pbuf[pl.ds(j0 - a8, 16)]
                        pids = jnp.where(lax.iota(jnp.int32, 16)
                                         < jnp.minimum(rem, 16), pids, e_pad)
                        nt = jnp.minimum(16, rem)

                        def tinit(l2, _i):
                            cB = jnp.zeros((16,), jnp.float32)
                            for jj in range(nd):
                                qv = (qbuf[l2, pl.ds(jj * 16, 16)]
                                      + abuf[l, pl.ds(jj * 16, 16)])
                                qbuf[l2, pl.ds(jj * 16, 16)] = qv
                                cB = cB + qv * bbuf[l, pl.ds(jj * 16, 16)]
                                state[pl.ds(l2 * stride + jj * 16, 16)] = (
                                    jnp.zeros((16,), jnp.float32))
                            cB = _allsum16(cB) * scale
                            state[pl.ds(l2 * stride + nd * 16, 16)] = (
                                jnp.zeros((16,), jnp.float32))
                            state[pl.ds(l2 * stride + (nd + 1) * 16, 16)] = (
                                jnp.zeros((16,), jnp.float32))
                            state[pl.ds(l2 * stride + (nd + 2) * 16, 16)] = cB
                            return _i

                        lax.fori_loop(0, nt, tinit, 0)

                        def ksuper(ks, _k):
                            kb0 = klo + ks * KCAP
                            for sb in range(KCAP // 16):
                                @pl.when(ks * KCAP + sb * 16 < kcnt)
                                def _():
                                    aa = kb0 + sb * 16
                                    aa8 = pl.multiple_of((aa // 8) * 8, 8)
                                    pltpu.sync_copy(
                                        srcs_h.at[pl.ds(aa8, 24)], sbuf)
                                    kids = sbuf[pl.ds(aa - aa8, 16)]
                                    pltpu.async_copy(
                                        pkvA_h.at[kids],
                                        kvc.at[pl.ds(sb * 16, 16)], sem2)
                            for sb in range(KCAP // 16):
                                @pl.when(ks * KCAP + sb * 16 < kcnt)
                                def _():
                                    pltpu.make_async_copy(
                                        pkvA_h.at[pl.ds(0, 16)],
                                        kvc.at[pl.ds(sb * 16, 16)], sem2).wait()

                            def target(l2, _t2):
                                qs = [qbuf[l2, pl.ds(jj * 16, 16)]
                                      for jj in range(nd)]
                                accs = [state[pl.ds(l2 * stride + jj * 16, 16)]
                                        for jj in range(nd)]
                                den = state[pl.ds(l2 * stride + nd * 16, 16)]
                                s0 = state[pl.ds(l2 * stride + (nd + 1) * 16, 16)]
                                cB = state[pl.ds(l2 * stride + (nd + 2) * 16, 16)]

                                def kbatch(kb, c2):
                                    def kone(l3, c3):
                                        accs3, den3, s03 = c3
                                        accs3 = list(accs3)
                                        kidx = ks * KCAP + kb * 16 + l3
                                        row = kb * 16 + l3
                                        parts = [jnp.zeros((16,), jnp.float32)
                                                 for _ in range(4)]
                                        for jj in range(nd):
                                            parts[jj % 4] = parts[jj % 4] + (
                                                qs[jj]
                                                * kvc[row, pl.ds(jj * 16, 16)])
                                        dot = ((parts[0] + parts[1])
                                               + (parts[2] + parts[3]))
                                        s = _allsum16(dot) * scale + cB
                                        s03 = jnp.where(kidx == 0, s, s03)
                                        wgt = jnp.exp(s - s03)
                                        den3 = den3 + wgt
                                        for jj in range(nd):
                                            accs3[jj] = accs3[jj] + wgt * kvc[
                                                row, pl.ds(d + jj * 16, 16)]
                                        return tuple(accs3), den3, s03

                                    nkeys = jnp.minimum(
                                        16, kcnt - ks * KCAP - kb * 16)
                                    return lax.fori_loop(0, nkeys, kone, c2)

                                nkb = jnp.minimum(
                                    (kcnt - ks * KCAP + 15) // 16, KCAP // 16)
                                accs, den, s0 = lax.fori_loop(
                                    0, nkb, kbatch, (tuple(accs), den, s0))
                                for jj in range(nd):
                                    state[pl.ds(l2 * stride + jj * 16, 16)] = accs[jj]
                                state[pl.ds(l2 * stride + nd * 16, 16)] = den
                                state[pl.ds(l2 * stride + (nd + 1) * 16, 16)] = s0
                                return _t2

                            lax.fori_loop(0, nt, target, 0)
                            return _k

                        lax.fori_loop(0, nks, ksuper, 0)

                        def tfin(l2, _f):
                            den = state[pl.ds(l2 * stride + nd * 16, 16)]
                            rden = jnp.where(den > 0.0, 1.0 / den, 0.0)
                            has = den > 0.0
                            for jj in range(nd):
                                acc = state[pl.ds(l2 * stride + jj * 16, 16)]
                                vB = bbuf[l, pl.ds(d + jj * 16, 16)]
                                obuf[l2, pl.ds(jj * 16, 16)] = (
                                    acc * rden + jnp.where(has, vB, 0.0))
                            return _f

                        lax.fori_loop(0, nt, tfin, 0)
                        def zfill(l2, _z):
                            for jj in range(nd):
                                obuf[l2, pl.ds(jj * 16, 16)] = jnp.zeros(
                                    (16,), jnp.float32)
                            return _z
                        lax.fori_loop(nt, 16, zfill, 0)
                        pltpu.async_copy(obuf, out_h.at[pids], sem1).wait()
                        return _t

                    lax.fori_loop(0, ntb, tbatch, 0)

                return _n

            lax.fori_loop(0, 16, node, 0)
            return _g

        lax.fori_loop(0, NPW // 16, group, 0)

    return k(pqA, pqB, pkvA, pkvB, srcs_pad, dsts_pad, perm_pad, inst, outst)


def kernel(x, edge_index, Wq1, Wk1, Wv1, g1, b1, Wq2, Wk2, Wv2, g2, b2, Wq3, Wk3, Wv3):
    n_nodes = x.shape[0]
    n_edges = edge_index.shape[1]
    src = edge_index[0].astype(jnp.int32)
    dst = edge_index[1].astype(jnp.int32)

    perm_d = jnp.argsort(dst)
    perm_s = jnp.argsort(src)
    dst_sorted = dst[perm_d]
    srcs_d = src[perm_d]
    dsts_s = dst[perm_s]
    pad0 = jnp.zeros((32,), jnp.int32)
    srcs_pad = jnp.concatenate([srcs_d, pad0])
    dsts_pad = jnp.concatenate([dsts_s, pad0])
    perm_pad = jnp.concatenate([perm_s.astype(jnp.int32),
                                jnp.full((32,), n_edges, jnp.int32)])
    vr = jnp.arange(INST_LEN, dtype=jnp.int32)
    inst = jnp.searchsorted(dst_sorted, vr).astype(jnp.int32)
    src_sorted = src[perm_s]
    outst = jnp.searchsorted(src_sorted, vr).astype(jnp.int32)

    q1, kv1 = _proj1(x, Wq1, jnp.concatenate([Wk1, Wv1], axis=1))
    o1 = _seg_attn_sc(q1, kv1, srcs_pad, inst, 3)[:n_nodes]
    q2, kv2 = _ln_proj(o1, g1, b1, Wq2, jnp.concatenate([Wk2, Wv2], axis=1))
    o2 = _seg_attn_sc(q2, kv2, srcs_pad, inst, 1)[:n_nodes]
    pqA, pqB, pkvA, pkvB = _ln_proj4(
        o2, g2, b2,
        Wq3[:256], Wq3[256:],
        jnp.concatenate([Wk3[:256], Wv3[:256]], axis=1),
        jnp.concatenate([Wk3[256:], Wv3[256:]], axis=1),
    )
    out = _line_attn_sc(pqA, pqB, pkvA, pkvB, srcs_pad, dsts_pad, perm_pad,
                        inst, outst)
    return out[:n_edges]


# speedup vs baseline: 23.4041x; 1.0005x over previous
import functools
import numpy as np
import jax
import jax.numpy as jnp
from jax import lax
from jax.experimental import pallas as pl
from jax.experimental.pallas import tpu as pltpu
from jax.experimental.pallas import tpu_sc as plsc

N_NODES_C = 10000
N_EDGES_C = 160000
ROW_BLK = 200

SC_NC = 2
SC_NS = 16
SC_NW = SC_NC * SC_NS
NPW = 320
NPAD = SC_NW * NPW
ST_LEN = 336
INST_LEN = NPW * (SC_NW - 1) + ST_LEN


def _lane_gather(x, idx):
    dn = lax.GatherDimensionNumbers(offset_dims=(), collapsed_slice_dims=(0,),
                                    start_index_map=(0,))
    return lax.gather(x, idx[:, None], dn, slice_sizes=(1,),
                      mode=lax.GatherScatterMode.PROMISE_IN_BOUNDS)


def _allsum16(x):
    i = lax.iota(jnp.int32, 16)
    for k in (1, 2, 4, 8):
        x = x + _lane_gather(x, i ^ k)
    return x


def _proj_body(x_ref, wq_ref, wkv_ref, q_ref, kv_ref):
    xb = x_ref[...]
    q_ref[...] = jnp.dot(xb, wq_ref[...], preferred_element_type=jnp.float32)
    kv_ref[...] = jnp.dot(xb, wkv_ref[...], preferred_element_type=jnp.float32)


def _proj1(x, Wq, Wkv):
    n, f = x.shape
    dq, dkv = Wq.shape[1], Wkv.shape[1]
    grid = n // ROW_BLK
    return pl.pallas_call(
        _proj_body,
        grid=(grid,),
        in_specs=[
            pl.BlockSpec((ROW_BLK, f), lambda i: (i, 0)),
            pl.BlockSpec((f, dq), lambda i: (0, 0)),
            pl.BlockSpec((f, dkv), lambda i: (0, 0)),
        ],
        out_specs=[
            pl.BlockSpec((ROW_BLK, dq), lambda i: (i, 0)),
            pl.BlockSpec((ROW_BLK, dkv), lambda i: (i, 0)),
        ],
        out_shape=[
            jax.ShapeDtypeStruct((n, dq), jnp.float32),
            jax.ShapeDtypeStruct((n, dkv), jnp.float32),
        ],
    )(x, Wq, Wkv)


def _ln_proj_body(h_ref, g_ref, b_ref, wq_ref, wkv_ref, q_ref, kv_ref):
    h = h_ref[...]
    mu = jnp.mean(h, axis=-1, keepdims=True)
    var = jnp.mean(jnp.square(h - mu), axis=-1, keepdims=True)
    hn = (h - mu) * lax.rsqrt(var + 1e-5) * g_ref[...] + b_ref[...]
    hn = jnp.maximum(hn, 0.0)
    q_ref[...] = jnp.dot(hn, wq_ref[...], preferred_element_type=jnp.float32)
    kv_ref[...] = jnp.dot(hn, wkv_ref[...], preferred_element_type=jnp.float32)


def _ln_proj(h, g, b, Wq, Wkv):
    n, f = h.shape
    dq, dkv = Wq.shape[1], Wkv.shape[1]
    grid = n // ROW_BLK
    return pl.pallas_call(
        _ln_proj_body,
        grid=(grid,),
        in_specs=[
            pl.BlockSpec((ROW_BLK, f), lambda i: (i, 0)),
            pl.BlockSpec((f,), lambda i: (0,)),
            pl.BlockSpec((f,), lambda i: (0,)),
            pl.BlockSpec((f, dq), lambda i: (0, 0)),
            pl.BlockSpec((f, dkv), lambda i: (0, 0)),
        ],
        out_specs=[
            pl.BlockSpec((ROW_BLK, dq), lambda i: (i, 0)),
            pl.BlockSpec((ROW_BLK, dkv), lambda i: (i, 0)),
        ],
        out_shape=[
            jax.ShapeDtypeStruct((n, dq), jnp.float32),
            jax.ShapeDtypeStruct((n, dkv), jnp.float32),
        ],
    )(h, g, b, Wq, Wkv)


def _ln_proj4_body(h_ref, g_ref, b_ref, wa_ref, wb_ref, wc_ref, wd_ref,
                   oa_ref, ob_ref, oc_ref, od_ref):
    h = h_ref[...]
    mu = jnp.mean(h, axis=-1, keepdims=True)
    var = jnp.mean(jnp.square(h - mu), axis=-1, keepdims=True)
    hn = (h - mu) * lax.rsqrt(var + 1e-5) * g_ref[...] + b_ref[...]
    hn = jnp.maximum(hn, 0.0)
    oa_ref[...] = jnp.dot(hn, wa_ref[...], preferred_element_type=jnp.float32)
    ob_ref[...] = jnp.dot(hn, wb_ref[...], preferred_element_type=jnp.float32)
    oc_ref[...] = jnp.dot(hn, wc_ref[...], preferred_element_type=jnp.float32)
    od_ref[...] = jnp.dot(hn, wd_ref[...], preferred_element_type=jnp.float32)


def _ln_proj4(h, g, b, Wa, Wb, Wc, Wd):
    n, f = h.shape
    dims = [W.shape[1] for W in (Wa, Wb, Wc, Wd)]
    grid = n // ROW_BLK
    return pl.pallas_call(
        _ln_proj4_body,
        grid=(grid,),
        in_specs=[
            pl.BlockSpec((ROW_BLK, f), lambda i: (i, 0)),
            pl.BlockSpec((f,), lambda i: (0,)),
            pl.BlockSpec((f,), lambda i: (0,)),
        ] + [pl.BlockSpec((f, d), lambda i: (0, 0)) for d in dims],
        out_specs=[pl.BlockSpec((ROW_BLK, d), lambda i: (i, 0)) for d in dims],
        out_shape=[jax.ShapeDtypeStruct((n, d), jnp.float32) for d in dims],
    )(h, g, b, Wa, Wb, Wc, Wd)


def _seg_attn_sc(q, kv, srcs_pad, inst, heads):
    n, dq = q.shape
    dkv = kv.shape[1]
    dh = dq // heads
    ncq = dh // 16
    scale = 1.0 / np.sqrt(float(dh))
    mesh = plsc.VectorSubcoreMesh(core_axis_name="c", subcore_axis_name="s")

    @functools.partial(
        pl.kernel, mesh=mesh,
        out_type=jax.ShapeDtypeStruct((NPAD, dq), jnp.float32),
        scratch_types=[
            pltpu.VMEM((ST_LEN,), jnp.int32),
            pltpu.VMEM((40,), jnp.int32),
            pltpu.VMEM((16, dq), jnp.float32),
            pltpu.VMEM((16, dkv), jnp.float32),
            pltpu.VMEM((16, dkv), jnp.float32),
            pltpu.VMEM((16, dq), jnp.float32),
            pltpu.SemaphoreType.DMA,
            pltpu.SemaphoreType.DMA,
            pltpu.SemaphoreType.DMA,
        ],
    )
    def k(q_h, kv_h, srcs_h, inst_h, out_h, st_v, ibuf, qbuf, kvbufA, kvbufB,
          obuf, semq, semA, semB):
        w = lax.axis_index("s") * SC_NC + lax.axis_index("c")
        base = w * NPW
        pltpu.sync_copy(inst_h.at[pl.ds(pl.multiple_of(base, 8), ST_LEN)], st_v)

        def group(t, _g):
            v0 = base + t * 16
            ids = jnp.minimum(v0 + lax.iota(jnp.int32, 16), n - 1)
            pltpu.async_copy(q_h.at[ids], qbuf, semq).wait()

            def node(l, _n):
                stv = st_v[pl.ds(t * 16 + l, 16)]
                lo = stv[0]
                hi = stv[1]
                cnt = hi - lo
                for jj in range(dq // 16):
                    obuf[l, pl.ds(jj * 16, 16)] = jnp.zeros((16,), jnp.float32)
                nb = (cnt + 15) // 16

                def fire(b, buf, sem):
                    a = lo + b * 16
                    a8 = pl.multiple_of((a // 8) * 8, 8)
                    pltpu.sync_copy(srcs_h.at[pl.ds(a8, 40)], ibuf)
                    ids_e = ibuf[pl.ds(a - a8, 16)]
                    pltpu.async_copy(kv_h.at[ids_e], buf, sem)

                def drain(buf, sem):
                    pltpu.make_async_copy(kv_h.at[pl.ds(0, 16)], buf, sem).wait()

                def consume(b, buf, carry):
                    def edge(l2, carry2):
                        dens2, s0s2 = carry2
                        j = b * 16 + l2
                        new_d, new_s = [], []
                        for h in range(heads):
                            parts = [jnp.zeros((16,), jnp.float32)
                                     for _ in range(4)]
                            for jj in range(ncq):
                                c0 = h * dh + jj * 16
                                parts[jj % 4] = parts[jj % 4] + (
                                    qbuf[l, pl.ds(c0, 16)]
                                    * buf[l2, pl.ds(c0, 16)])
                            dot = (parts[0] + parts[1]) + (parts[2] + parts[3])
                            s = _allsum16(dot) * scale
                            s0h = jnp.where(j == 0, s, s0s2[h])
                            wgt = jnp.exp(s - s0h)
                            new_d.append(dens2[h] + wgt)
                            new_s.append(s0h)
                            for jj in range(ncq):
                                c0 = h * dh + jj * 16
                                obuf[l, pl.ds(c0, 16)] = (
                                    obuf[l, pl.ds(c0, 16)]
                                    + wgt * buf[l2, pl.ds(dq + c0, 16)])
                        return tuple(new_d), tuple(new_s)

                    return lax.fori_loop(0, jnp.minimum(16, cnt - b * 16),
                                         edge, carry)

                @pl.when(nb > 0)
                def _():
                    fire(0, kvbufA, semA)

                def pair(i, carry):
                    b0 = 2 * i
                    b1 = 2 * i + 1
                    a = lo + b1 * 16
                    a8 = pl.multiple_of((a // 8) * 8, 8)

                    @pl.when(b1 < nb)
                    def _():
                        pltpu.sync_copy(srcs_h.at[pl.ds(a8, 40)], ibuf)
                        ids_e = ibuf[pl.ds(a - a8, 16)]
                        pltpu.async_copy(kv_h.at[ids_e], kvbufB, semB)
                    drain(kvbufA, semA)
                    carry = consume(b0, kvbufA, carry)

                    @pl.when(b1 + 1 < nb)
                    def _():
                        ids_e = ibuf[pl.ds(a - a8 + 16, 16)]
                        pltpu.async_copy(kv_h.at[ids_e], kvbufA, semA)

                    @pl.when(b1 < nb)
                    def _():
                        drain(kvbufB, semB)
                    carry = consume(b1, kvbufB, carry)
                    return carry

                zero = tuple(jnp.zeros((16,), jnp.float32) for _ in range(heads))
                dens, _ = lax.fori_loop(0, (nb + 1) // 2, pair, (zero, zero))
                for h in range(heads):
                    rden = jnp.where(dens[h] > 0.0, 1.0 / dens[h], 0.0)
                    for jj in range(ncq):
                        c0 = h * dh + jj * 16
                        obuf[l, pl.ds(c0, 16)] = obuf[l, pl.ds(c0, 16)] * rden
                return _n

            lax.fori_loop(0, 16, node, 0)
            pltpu.sync_copy(obuf, out_h.at[pl.ds(pl.multiple_of(v0, 8), 16)])
            return _g

        lax.fori_loop(0, NPW // 16, group, 0)

    return k(q, kv, srcs_pad, inst)


def _line_attn_sc(pqA, pqB, pkvA, pkvB, srcs_pad, dsts_pad, perm_pad, inst, outst):
    n, d = pqA.shape
    nd = d // 16
    e_pad = perm_pad.shape[0] - 64
    scale = 1.0 / np.sqrt(float(d))
    KCAP = 64
    mesh = plsc.VectorSubcoreMesh(core_axis_name="c", subcore_axis_name="s")

    @functools.partial(
        pl.kernel, mesh=mesh,
        out_type=jax.ShapeDtypeStruct((e_pad + 16, d), jnp.float32),
        scratch_types=[
            pltpu.VMEM((ST_LEN,), jnp.int32),
            pltpu.VMEM((ST_LEN,), jnp.int32),
            pltpu.VMEM((24,), jnp.int32),
            pltpu.VMEM((24,), jnp.int32),
            pltpu.VMEM((24,), jnp.int32),
            pltpu.VMEM((16, d), jnp.float32),
            pltpu.VMEM((16, 2 * d), jnp.float32),
            pltpu.VMEM((16, d), jnp.float32),
            pltpu.VMEM((KCAP, 2 * d), jnp.float32),
            pltpu.VMEM((16, d), jnp.float32),
            pltpu.VMEM((16 * (nd + 3) * 16,), jnp.float32),
            pltpu.SemaphoreType.DMA,
            pltpu.SemaphoreType.DMA,
        ],
    )
    def k(pqA_h, pqB_h, pkvA_h, pkvB_h, srcs_h, dsts_h, perm_h, inst_h,
          outst_h, out_h, st_in, st_out, sbuf, dbuf, pbuf, abuf, bbuf, qbuf,
          kvc, obuf, state, sem1, sem2):
        w = lax.axis_index("s") * SC_NC + lax.axis_index("c")
        base = w * NPW
        pltpu.sync_copy(inst_h.at[pl.ds(pl.multiple_of(base, 8), ST_LEN)], st_in)
        pltpu.sync_copy(outst_h.at[pl.ds(pl.multiple_of(base, 8), ST_LEN)], st_out)
        stride = (nd + 3) * 16

        def group(t, _g):
            v0 = base + t * 16
            ids = jnp.minimum(v0 + lax.iota(jnp.int32, 16), n - 1)
            pltpu.async_copy(pqA_h.at[ids], abuf, sem1).wait()
            pltpu.async_copy(pkvB_h.at[ids], bbuf, sem1).wait()

            def node(l, _n):
                sti = st_in[pl.ds(t * 16 + l, 16)]
                klo = sti[0]
                kcnt = sti[1] - klo
                sto = st_out[pl.ds(t * 16 + l, 16)]
                jlo = sto[0]
                ocnt = sto[1] - jlo

                @pl.when(ocnt > 0)
                def _():
                    nks = (kcnt + KCAP - 1) // KCAP
                    ntb = (ocnt + 15) // 16

                    def tbatch(tb, _t):
                        j0 = jlo + tb * 16
                        rem = ocnt - tb * 16
                        a8 = pl.multiple_of((j0 // 8) * 8, 8)
                        pltpu.sync_copy(dsts_h.at[pl.ds(a8, 24)], dbuf)
                        tids = dbuf[pl.ds(j0 - a8, 16)]
                        pltpu.async_copy(pqB_h.at[tids], qbuf, sem1).wait()
                        pltpu.sync_copy(perm_h.at[pl.ds(a8, 24)], pbuf)
                        pids = pbuf[pl.ds(j0 - a8, 16)]
                        pids = jnp.where(lax.iota(jnp.int32, 16)
                                         < jnp.minimum(rem, 16), pids, e_pad)
                        nt = jnp.minimum(16, rem)

                        def tinit(l2, _i):
                            cB = jnp.zeros((16,), jnp.float32)
                            for jj in range(nd):
                                qv = (qbuf[l2, pl.ds(jj * 16, 16)]
                                      + abuf[l, pl.ds(jj * 16, 16)])
                                qbuf[l2, pl.ds(jj * 16, 16)] = qv
                                cB = cB + qv * bbuf[l, pl.ds(jj * 16, 16)]
                                state[pl.ds(l2 * stride + jj * 16, 16)] = (
                                    jnp.zeros((16,), jnp.float32))
                            cB = _allsum16(cB) * scale
                            state[pl.ds(l2 * stride + nd * 16, 16)] = (
                                jnp.zeros((16,), jnp.float32))
                            state[pl.ds(l2 * stride + (nd + 1) * 16, 16)] = (
                                jnp.zeros((16,), jnp.float32))
                            state[pl.ds(l2 * stride + (nd + 2) * 16, 16)] = cB
                            return _i

                        lax.fori_loop(0, nt, tinit, 0)

                        def ksuper(ks, _k):
                            kb0 = klo + ks * KCAP
                            for sb in range(KCAP // 16):
                                @pl.when(ks * KCAP + sb * 16 < kcnt)
                                def _():
                                    aa = kb0 + sb * 16
                                    aa8 = pl.multiple_of((aa // 8) * 8, 8)
                                    pltpu.sync_copy(
                                        srcs_h.at[pl.ds(aa8, 24)], sbuf)
                                    kids = sbuf[pl.ds(aa - aa8, 16)]
                                    pltpu.async_copy(
                                        pkvA_h.at[kids],
                                        kvc.at[pl.ds(sb * 16, 16)], sem2)
                            for sb in range(KCAP // 16):
                                @pl.when(ks * KCAP + sb * 16 < kcnt)
                                def _():
                                    pltpu.make_async_copy(
                                        pkvA_h.at[pl.ds(0, 16)],
                                        kvc.at[pl.ds(sb * 16, 16)], sem2).wait()

                            def target(l2, _t2):
                                qs = [qbuf[l2, pl.ds(jj * 16, 16)]
                                      for jj in range(nd)]
                                accs = [state[pl.ds(l2 * stride + jj * 16, 16)]
                                        for jj in range(nd)]
                                den = state[pl.ds(l2 * stride + nd * 16, 16)]
                                s0 = state[pl.ds(l2 * stride + (nd + 1) * 16, 16)]
                                cB = state[pl.ds(l2 * stride + (nd + 2) * 16, 16)]

                                def kbatch(kb, c2):
                                    def kone(l3, c3):
                                        accs3, den3, s03 = c3
                                        accs3 = list(accs3)
                                        kidx = ks * KCAP + kb * 16 + l3
                                        row = kb * 16 + l3
                                        parts = [jnp.zeros((16,), jnp.float32)
                                                 for _ in range(4)]
                                        for jj in range(nd):
                                            parts[jj % 4] = parts[jj % 4] + (
                                                qs[jj]
                                                * kvc[row, pl.ds(jj * 16, 16)])
                                        dot = ((parts[0] + parts[1])
                                               + (parts[2] + parts[3]))
                                        s = _allsum16(dot) * scale + cB
                                        s03 = jnp.where(kidx == 0, s, s03)
                                        wgt = jnp.exp(s - s03)
                                        den3 = den3 + wgt
                                        for jj in range(nd):
                                            accs3[jj] = accs3[jj] + wgt * kvc[
                                                row, pl.ds(d + jj * 16, 16)]
                                        return tuple(accs3), den3, s03

                                    nkeys = jnp.minimum(
                                        16, kcnt - ks * KCAP - kb * 16)
                                    return lax.fori_loop(0, nkeys, kone, c2)

                                nkb = jnp.minimum(
                                    (kcnt - ks * KCAP + 15) // 16, KCAP // 16)
                                accs, den, s0 = lax.fori_loop(
                                    0, nkb, kbatch, (tuple(accs), den, s0))
                                for jj in range(nd):
                                    state[pl.ds(l2 * stride + jj * 16, 16)] = accs[jj]
                                state[pl.ds(l2 * stride + nd * 16, 16)] = den
                                state[pl.ds(l2 * stride + (nd + 1) * 16, 16)] = s0
                                return _t2

                            lax.fori_loop(0, nt, target, 0)
                            return _k

                        lax.fori_loop(0, nks, ksuper, 0)

                        def tfin(l2, _f):
                            den = state[pl.ds(l2 * stride + nd * 16, 16)]
                            rden = jnp.where(den > 0.0, 1.0 / den, 0.0)
                            has = den > 0.0
                            for jj in range(nd):
                                acc = state[pl.ds(l2 * stride + jj * 16, 16)]
                                vB = bbuf[l, pl.ds(d + jj * 16, 16)]
                                obuf[l2, pl.ds(jj * 16, 16)] = (
                                    acc * rden + jnp.where(has, vB, 0.0))
                            return _f

                        lax.fori_loop(0, nt, tfin, 0)
                        def zfill(l2, _z):
                            for jj in range(nd):
                                obuf[l2, pl.ds(jj * 16, 16)] = jnp.zeros(
                                    (16,), jnp.float32)
                            return _z
                        lax.fori_loop(nt, 16, zfill, 0)
                        pltpu.async_copy(obuf, out_h.at[pids], sem1).wait()
                        return _t

                    lax.fori_loop(0, ntb, tbatch, 0)

                return _n

            lax.fori_loop(0, 16, node, 0)
            return _g

        lax.fori_loop(0, NPW // 16, group, 0)

    return k(pqA, pqB, pkvA, pkvB, srcs_pad, dsts_pad, perm_pad, inst, outst)


def kernel(x, edge_index, Wq1, Wk1, Wv1, g1, b1, Wq2, Wk2, Wv2, g2, b2, Wq3, Wk3, Wv3):
    n_nodes = x.shape[0]
    n_edges = edge_index.shape[1]
    src = edge_index[0].astype(jnp.int32)
    dst = edge_index[1].astype(jnp.int32)

    perm_d = jnp.argsort(dst)
    perm_s = jnp.argsort(src)
    dst_sorted = dst[perm_d]
    srcs_d = src[perm_d]
    dsts_s = dst[perm_s]
    pad0 = jnp.zeros((64,), jnp.int32)
    srcs_pad = jnp.concatenate([srcs_d, pad0])
    dsts_pad = jnp.concatenate([dsts_s, pad0])
    perm_pad = jnp.concatenate([perm_s.astype(jnp.int32),
                                jnp.full((64,), n_edges, jnp.int32)])
    vr = jnp.arange(INST_LEN, dtype=jnp.int32)
    inst = jnp.searchsorted(dst_sorted, vr).astype(jnp.int32)
    src_sorted = src[perm_s]
    outst = jnp.searchsorted(src_sorted, vr).astype(jnp.int32)

    q1, kv1 = _proj1(x, Wq1, jnp.concatenate([Wk1, Wv1], axis=1))
    o1 = _seg_attn_sc(q1, kv1, srcs_pad, inst, 3)[:n_nodes]
    q2, kv2 = _ln_proj(o1, g1, b1, Wq2, jnp.concatenate([Wk2, Wv2], axis=1))
    o2 = _seg_attn_sc(q2, kv2, srcs_pad, inst, 1)[:n_nodes]
    pqA, pqB, pkvA, pkvB = _ln_proj4(
        o2, g2, b2,
        Wq3[:256], Wq3[256:],
        jnp.concatenate([Wk3[:256], Wv3[:256]], axis=1),
        jnp.concatenate([Wk3[256:], Wv3[256:]], axis=1),
    )
    out = _line_attn_sc(pqA, pqB, pkvA, pkvB, srcs_pad, dsts_pad, perm_pad,
                        inst, outst)
    return out[:n_edges]


# speedup vs baseline: 24.3348x; 1.0398x over previous
import functools
import numpy as np
import jax
import jax.numpy as jnp
from jax import lax
from jax.experimental import pallas as pl
from jax.experimental.pallas import tpu as pltpu
from jax.experimental.pallas import tpu_sc as plsc

N_NODES_C = 10000
N_EDGES_C = 160000
ROW_BLK = 200

SC_NC = 2
SC_NS = 16
SC_NW = SC_NC * SC_NS
NPW = 320
NPAD = SC_NW * NPW
ST_LEN = 336
INST_LEN = NPW * (SC_NW - 1) + ST_LEN


def _lane_gather(x, idx):
    dn = lax.GatherDimensionNumbers(offset_dims=(), collapsed_slice_dims=(0,),
                                    start_index_map=(0,))
    return lax.gather(x, idx[:, None], dn, slice_sizes=(1,),
                      mode=lax.GatherScatterMode.PROMISE_IN_BOUNDS)


def _allsum16(x):
    i = lax.iota(jnp.int32, 16)
    for k in (1, 2, 4, 8):
        x = x + _lane_gather(x, i ^ k)
    return x


def _proj_body(x_ref, wq_ref, wkv_ref, q_ref, kv_ref):
    xb = x_ref[...]
    q_ref[...] = jnp.dot(xb, wq_ref[...], preferred_element_type=jnp.float32)
    kv_ref[...] = jnp.dot(xb, wkv_ref[...], preferred_element_type=jnp.float32)


def _proj1(x, Wq, Wkv):
    n, f = x.shape
    dq, dkv = Wq.shape[1], Wkv.shape[1]
    grid = n // ROW_BLK
    return pl.pallas_call(
        _proj_body,
        grid=(grid,),
        in_specs=[
            pl.BlockSpec((ROW_BLK, f), lambda i: (i, 0)),
            pl.BlockSpec((f, dq), lambda i: (0, 0)),
            pl.BlockSpec((f, dkv), lambda i: (0, 0)),
        ],
        out_specs=[
            pl.BlockSpec((ROW_BLK, dq), lambda i: (i, 0)),
            pl.BlockSpec((ROW_BLK, dkv), lambda i: (i, 0)),
        ],
        out_shape=[
            jax.ShapeDtypeStruct((n, dq), jnp.float32),
            jax.ShapeDtypeStruct((n, dkv), jnp.float32),
        ],
    )(x, Wq, Wkv)


def _ln_proj_body(h_ref, g_ref, b_ref, wq_ref, wkv_ref, q_ref, kv_ref):
    h = h_ref[...]
    mu = jnp.mean(h, axis=-1, keepdims=True)
    var = jnp.mean(jnp.square(h - mu), axis=-1, keepdims=True)
    hn = (h - mu) * lax.rsqrt(var + 1e-5) * g_ref[...] + b_ref[...]
    hn = jnp.maximum(hn, 0.0)
    q_ref[...] = jnp.dot(hn, wq_ref[...], preferred_element_type=jnp.float32)
    kv_ref[...] = jnp.dot(hn, wkv_ref[...], preferred_element_type=jnp.float32)


def _ln_proj(h, g, b, Wq, Wkv):
    n, f = h.shape
    dq, dkv = Wq.shape[1], Wkv.shape[1]
    grid = n // ROW_BLK
    return pl.pallas_call(
        _ln_proj_body,
        grid=(grid,),
        in_specs=[
            pl.BlockSpec((ROW_BLK, f), lambda i: (i, 0)),
            pl.BlockSpec((f,), lambda i: (0,)),
            pl.BlockSpec((f,), lambda i: (0,)),
            pl.BlockSpec((f, dq), lambda i: (0, 0)),
            pl.BlockSpec((f, dkv), lambda i: (0, 0)),
        ],
        out_specs=[
            pl.BlockSpec((ROW_BLK, dq), lambda i: (i, 0)),
            pl.BlockSpec((ROW_BLK, dkv), lambda i: (i, 0)),
        ],
        out_shape=[
            jax.ShapeDtypeStruct((n, dq), jnp.float32),
            jax.ShapeDtypeStruct((n, dkv), jnp.float32),
        ],
    )(h, g, b, Wq, Wkv)


def _ln_proj4_body(h_ref, g_ref, b_ref, wa_ref, wb_ref, wc_ref, wd_ref,
                   oa_ref, ob_ref, oc_ref, od_ref):
    h = h_ref[...]
    mu = jnp.mean(h, axis=-1, keepdims=True)
    var = jnp.mean(jnp.square(h - mu), axis=-1, keepdims=True)
    hn = (h - mu) * lax.rsqrt(var + 1e-5) * g_ref[...] + b_ref[...]
    hn = jnp.maximum(hn, 0.0)
    oa_ref[...] = jnp.dot(hn, wa_ref[...], preferred_element_type=jnp.float32)
    ob_ref[...] = jnp.dot(hn, wb_ref[...], preferred_element_type=jnp.float32)
    oc_ref[...] = jnp.dot(hn, wc_ref[...], preferred_element_type=jnp.float32)
    od_ref[...] = jnp.dot(hn, wd_ref[...], preferred_element_type=jnp.float32)


def _ln_proj4(h, g, b, Wa, Wb, Wc, Wd):
    n, f = h.shape
    dims = [W.shape[1] for W in (Wa, Wb, Wc, Wd)]
    grid = n // ROW_BLK
    return pl.pallas_call(
        _ln_proj4_body,
        grid=(grid,),
        in_specs=[
            pl.BlockSpec((ROW_BLK, f), lambda i: (i, 0)),
            pl.BlockSpec((f,), lambda i: (0,)),
            pl.BlockSpec((f,), lambda i: (0,)),
        ] + [pl.BlockSpec((f, d), lambda i: (0, 0)) for d in dims],
        out_specs=[pl.BlockSpec((ROW_BLK, d), lambda i: (i, 0)) for d in dims],
        out_shape=[jax.ShapeDtypeStruct((n, d), jnp.float32) for d in dims],
    )(h, g, b, Wa, Wb, Wc, Wd)


def _seg_attn_sc(q, kv, srcs_pad, inst, heads):
    n, dq = q.shape
    dkv = kv.shape[1]
    dh = dq // heads
    ncq = dh // 16
    scale = 1.0 / np.sqrt(float(dh))
    mesh = plsc.VectorSubcoreMesh(core_axis_name="c", subcore_axis_name="s")

    @functools.partial(
        pl.kernel, mesh=mesh,
        out_type=jax.ShapeDtypeStruct((NPAD, dq), jnp.float32),
        scratch_types=[
            pltpu.VMEM((ST_LEN,), jnp.int32),
            pltpu.VMEM((40,), jnp.int32),
            pltpu.VMEM((16, dq), jnp.float32),
            pltpu.VMEM((16, dkv), jnp.float32),
            pltpu.VMEM((16, dkv), jnp.float32),
            pltpu.VMEM((16, dq), jnp.float32),
            pltpu.SemaphoreType.DMA,
            pltpu.SemaphoreType.DMA,
            pltpu.SemaphoreType.DMA,
        ],
    )
    def k(q_h, kv_h, srcs_h, inst_h, out_h, st_v, ibuf, qbuf, kvbufA, kvbufB,
          obuf, semq, semA, semB):
        w = lax.axis_index("s") * SC_NC + lax.axis_index("c")
        base = w * NPW
        pltpu.sync_copy(inst_h.at[pl.ds(pl.multiple_of(base, 8), ST_LEN)], st_v)

        def group(t, _g):
            v0 = base + t * 16
            ids = jnp.minimum(v0 + lax.iota(jnp.int32, 16), n - 1)
            pltpu.async_copy(q_h.at[ids], qbuf, semq).wait()

            def node(l, _n):
                stv = st_v[pl.ds(t * 16 + l, 16)]
                lo = stv[0]
                hi = stv[1]
                cnt = hi - lo
                for jj in range(dq // 16):
                    obuf[l, pl.ds(jj * 16, 16)] = jnp.zeros((16,), jnp.float32)
                nb = (cnt + 15) // 16

                def fire(b, buf, sem):
                    a = lo + b * 16
                    a8 = pl.multiple_of((a // 8) * 8, 8)
                    pltpu.sync_copy(srcs_h.at[pl.ds(a8, 40)], ibuf)
                    ids_e = ibuf[pl.ds(a - a8, 16)]
                    pltpu.async_copy(kv_h.at[ids_e], buf, sem)

                def drain(buf, sem):
                    pltpu.make_async_copy(kv_h.at[pl.ds(0, 16)], buf, sem).wait()

                def consume(b, buf, carry):
                    def edge(l2, carry2):
                        dens2, s0s2 = carry2
                        j = b * 16 + l2
                        new_d, new_s = [], []
                        for h in range(heads):
                            parts = [jnp.zeros((16,), jnp.float32)
                                     for _ in range(4)]
                            for jj in range(ncq):
                                c0 = h * dh + jj * 16
                                parts[jj % 4] = parts[jj % 4] + (
                                    qbuf[l, pl.ds(c0, 16)]
                                    * buf[l2, pl.ds(c0, 16)])
                            dot = (parts[0] + parts[1]) + (parts[2] + parts[3])
                            s = _allsum16(dot) * scale
                            s0h = jnp.where(j == 0, s, s0s2[h])
                            wgt = jnp.exp(s - s0h)
                            new_d.append(dens2[h] + wgt)
                            new_s.append(s0h)
                            for jj in range(ncq):
                                c0 = h * dh + jj * 16
                                obuf[l, pl.ds(c0, 16)] = (
                                    obuf[l, pl.ds(c0, 16)]
                                    + wgt * buf[l2, pl.ds(dq + c0, 16)])
                        return tuple(new_d), tuple(new_s)

                    return lax.fori_loop(0, jnp.minimum(16, cnt - b * 16),
                                         edge, carry)

                @pl.when(nb > 0)
                def _():
                    fire(0, kvbufA, semA)

                def pair(i, carry):
                    b0 = 2 * i
                    b1 = 2 * i + 1
                    a = lo + b1 * 16
                    a8 = pl.multiple_of((a // 8) * 8, 8)

                    @pl.when(b1 < nb)
                    def _():
                        pltpu.sync_copy(srcs_h.at[pl.ds(a8, 40)], ibuf)
                        ids_e = ibuf[pl.ds(a - a8, 16)]
                        pltpu.async_copy(kv_h.at[ids_e], kvbufB, semB)
                    drain(kvbufA, semA)
                    carry = consume(b0, kvbufA, carry)

                    @pl.when(b1 + 1 < nb)
                    def _():
                        ids_e = ibuf[pl.ds(a - a8 + 16, 16)]
                        pltpu.async_copy(kv_h.at[ids_e], kvbufA, semA)

                    @pl.when(b1 < nb)
                    def _():
                        drain(kvbufB, semB)
                    carry = consume(b1, kvbufB, carry)
                    return carry

                zero = tuple(jnp.zeros((16,), jnp.float32) for _ in range(heads))
                dens, _ = lax.fori_loop(0, (nb + 1) // 2, pair, (zero, zero))
                for h in range(heads):
                    rden = jnp.where(dens[h] > 0.0, 1.0 / dens[h], 0.0)
                    for jj in range(ncq):
                        c0 = h * dh + jj * 16
                        obuf[l, pl.ds(c0, 16)] = obuf[l, pl.ds(c0, 16)] * rden
                return _n

            lax.fori_loop(0, 16, node, 0)
            pltpu.sync_copy(obuf, out_h.at[pl.ds(pl.multiple_of(v0, 8), 16)])
            return _g

        lax.fori_loop(0, NPW // 16, group, 0)

    return k(q, kv, srcs_pad, inst)


def _line_attn_sc(pqA, pqB, pkvA, pkvB, srcs_pad, dsts_pad, perm_pad, inst, outst):
    n, d = pqA.shape
    nd = d // 16
    e_pad = perm_pad.shape[0] - 64
    scale = 1.0 / np.sqrt(float(d))
    KCAP = 64
    mesh = plsc.VectorSubcoreMesh(core_axis_name="c", subcore_axis_name="s")

    @functools.partial(
        pl.kernel, mesh=mesh,
        out_type=jax.ShapeDtypeStruct((e_pad + 16, d), jnp.float32),
        scratch_types=[
            pltpu.VMEM((ST_LEN,), jnp.int32),
            pltpu.VMEM((ST_LEN,), jnp.int32),
            pltpu.VMEM((24,), jnp.int32),
            pltpu.VMEM((24,), jnp.int32),
            pltpu.VMEM((24,), jnp.int32),
            pltpu.VMEM((16, d), jnp.float32),
            pltpu.VMEM((16, 2 * d), jnp.float32),
            pltpu.VMEM((16, d), jnp.float32),
            pltpu.VMEM((KCAP, 2 * d), jnp.float32),
            pltpu.VMEM((16, d), jnp.float32),
            pltpu.VMEM((16 * (nd + 3) * 16,), jnp.float32),
            pltpu.SemaphoreType.DMA,
            pltpu.SemaphoreType.DMA,
        ],
    )
    def k(pqA_h, pqB_h, pkvA_h, pkvB_h, srcs_h, dsts_h, perm_h, inst_h,
          outst_h, out_h, st_in, st_out, sbuf, dbuf, pbuf, abuf, bbuf, qbuf,
          kvc, obuf, state, sem1, sem2):
        w = lax.axis_index("s") * SC_NC + lax.axis_index("c")
        base = w * NPW
        pltpu.sync_copy(inst_h.at[pl.ds(pl.multiple_of(base, 8), ST_LEN)], st_in)
        pltpu.sync_copy(outst_h.at[pl.ds(pl.multiple_of(base, 8), ST_LEN)], st_out)
        stride = (nd + 3) * 16

        def group(t, _g):
            v0 = base + t * 16
            ids = jnp.minimum(v0 + lax.iota(jnp.int32, 16), n - 1)
            pltpu.async_copy(pqA_h.at[ids], abuf, sem1).wait()
            pltpu.async_copy(pkvB_h.at[ids], bbuf, sem1).wait()

            def node(l, _n):
                sti = st_in[pl.ds(t * 16 + l, 16)]
                klo = sti[0]
                kcnt = sti[1] - klo
                sto = st_out[pl.ds(t * 16 + l, 16)]
                jlo = sto[0]
                ocnt = sto[1] - jlo

                @pl.when(ocnt > 0)
                def _():
                    nks = (kcnt + KCAP - 1) // KCAP
                    ntb = (ocnt + 15) // 16

                    def kfire(ks):
                        kb0 = klo + ks * KCAP
                        for sb in range(KCAP // 16):
                            @pl.when(ks * KCAP + sb * 16 < kcnt)
                            def _():
                                aa = kb0 + sb * 16
                                aa8 = pl.multiple_of((aa // 8) * 8, 8)
                                pltpu.sync_copy(srcs_h.at[pl.ds(aa8, 24)], sbuf)
                                kids = sbuf[pl.ds(aa - aa8, 16)]
                                pltpu.async_copy(pkvA_h.at[kids],
                                                 kvc.at[pl.ds(sb * 16, 16)],
                                                 sem2)

                    def tbatch(tb, _t):
                        j0 = jlo + tb * 16
                        rem = ocnt - tb * 16
                        a8 = pl.multiple_of((j0 // 8) * 8, 8)
                        pltpu.sync_copy(dsts_h.at[pl.ds(a8, 24)], dbuf)
                        tids = dbuf[pl.ds(j0 - a8, 16)]
                        pltpu.async_copy(pqB_h.at[tids], qbuf, sem1)
                        kfire(0)
                        pltpu.sync_copy(perm_h.at[pl.ds(a8, 24)], pbuf)
                        pids = pbuf[pl.ds(j0 - a8, 16)]
                        pids = jnp.where(lax.iota(jnp.int32, 16)
                                         < jnp.minimum(rem, 16), pids, e_pad)
                        nt = jnp.minimum(16, rem)
                        pltpu.make_async_copy(pqB_h.at[pl.ds(0, 16)], qbuf,
                                              sem1).wait()

                        def tinit(l2, _i):
                            cB = jnp.zeros((16,), jnp.float32)
                            for jj in range(nd):
                                qv = (qbuf[l2, pl.ds(jj * 16, 16)]
                                      + abuf[l, pl.ds(jj * 16, 16)])
                                qbuf[l2, pl.ds(jj * 16, 16)] = qv
                                cB = cB + qv * bbuf[l, pl.ds(jj * 16, 16)]
                                state[pl.ds(l2 * stride + jj * 16, 16)] = (
                                    jnp.zeros((16,), jnp.float32))
                            cB = _allsum16(cB) * scale
                            state[pl.ds(l2 * stride + nd * 16, 16)] = (
                                jnp.zeros((16,), jnp.float32))
                            state[pl.ds(l2 * stride + (nd + 1) * 16, 16)] = (
                                jnp.zeros((16,), jnp.float32))
                            state[pl.ds(l2 * stride + (nd + 2) * 16, 16)] = cB
                            return _i

                        lax.fori_loop(0, nt, tinit, 0)

                        def ksuper(ks, _k):
                            for sb in range(KCAP // 16):
                                @pl.when(ks * KCAP + sb * 16 < kcnt)
                                def _():
                                    pltpu.make_async_copy(
                                        pkvA_h.at[pl.ds(0, 16)],
                                        kvc.at[pl.ds(sb * 16, 16)], sem2).wait()

                            def target(l2, _t2):
                                qs = [qbuf[l2, pl.ds(jj * 16, 16)]
                                      for jj in range(nd)]
                                accs = [state[pl.ds(l2 * stride + jj * 16, 16)]
                                        for jj in range(nd)]
                                den = state[pl.ds(l2 * stride + nd * 16, 16)]
                                s0 = state[pl.ds(l2 * stride + (nd + 1) * 16, 16)]
                                cB = state[pl.ds(l2 * stride + (nd + 2) * 16, 16)]

                                def kbatch(kb, c2):
                                    def kone(l3, c3):
                                        accs3, den3, s03 = c3
                                        accs3 = list(accs3)
                                        kidx = ks * KCAP + kb * 16 + l3
                                        row = kb * 16 + l3
                                        parts = [jnp.zeros((16,), jnp.float32)
                                                 for _ in range(4)]
                                        for jj in range(nd):
                                            parts[jj % 4] = parts[jj % 4] + (
                                                qs[jj]
                                                * kvc[row, pl.ds(jj * 16, 16)])
                                        dot = ((parts[0] + parts[1])
                                               + (parts[2] + parts[3]))
                                        s = _allsum16(dot) * scale + cB
                                        s03 = jnp.where(kidx == 0, s, s03)
                                        wgt = jnp.exp(s - s03)
                                        den3 = den3 + wgt
                                        for jj in range(nd):
                                            accs3[jj] = accs3[jj] + wgt * kvc[
                                                row, pl.ds(d + jj * 16, 16)]
                                        return tuple(accs3), den3, s03

                                    nkeys = jnp.minimum(
                                        16, kcnt - ks * KCAP - kb * 16)
                                    return lax.fori_loop(0, nkeys, kone, c2)

                                nkb = jnp.minimum(
                                    (kcnt - ks * KCAP + 15) // 16, KCAP // 16)
                                accs, den, s0 = lax.fori_loop(
                                    0, nkb, kbatch, (tuple(accs), den, s0))
                                for jj in range(nd):
                                    state[pl.ds(l2 * stride + jj * 16, 16)] = accs[jj]
                                state[pl.ds(l2 * stride + nd * 16, 16)] = den
                                state[pl.ds(l2 * stride + (nd + 1) * 16, 16)] = s0
                                return _t2

                            lax.fori_loop(0, nt, target, 0)
                            kfire(ks + 1)
                            return _k

                        lax.fori_loop(0, nks, ksuper, 0)

                        def tfin(l2, _f):
                            den = state[pl.ds(l2 * stride + nd * 16, 16)]
                            rden = jnp.where(den > 0.0, 1.0 / den, 0.0)
                            has = den > 0.0
                            for jj in range(nd):
                                acc = state[pl.ds(l2 * stride + jj * 16, 16)]
                                vB = bbuf[l, pl.ds(d + jj * 16, 16)]
                                obuf[l2, pl.ds(jj * 16, 16)] = (
                                    acc * rden + jnp.where(has, vB, 0.0))
                            return _f

                        lax.fori_loop(0, nt, tfin, 0)
                        def zfill(l2, _z):
                            for jj in range(nd):
                                obuf[l2, pl.ds(jj * 16, 16)] = jnp.zeros(
                                    (16,), jnp.float32)
                            return _z
                        lax.fori_loop(nt, 16, zfill, 0)
                        pltpu.async_copy(obuf, out_h.at[pids], sem1).wait()
                        return _t

                    lax.fori_loop(0, ntb, tbatch, 0)

                return _n

            lax.fori_loop(0, 16, node, 0)
            return _g

        lax.fori_loop(0, NPW // 16, group, 0)

    return k(pqA, pqB, pkvA, pkvB, srcs_pad, dsts_pad, perm_pad, inst, outst)


def kernel(x, edge_index, Wq1, Wk1, Wv1, g1, b1, Wq2, Wk2, Wv2, g2, b2, Wq3, Wk3, Wv3):
    n_nodes = x.shape[0]
    n_edges = edge_index.shape[1]
    src = edge_index[0].astype(jnp.int32)
    dst = edge_index[1].astype(jnp.int32)

    perm_d = jnp.argsort(dst)
    perm_s = jnp.argsort(src)
    dst_sorted = dst[perm_d]
    srcs_d = src[perm_d]
    dsts_s = dst[perm_s]
    pad0 = jnp.zeros((64,), jnp.int32)
    srcs_pad = jnp.concatenate([srcs_d, pad0])
    dsts_pad = jnp.concatenate([dsts_s, pad0])
    perm_pad = jnp.concatenate([perm_s.astype(jnp.int32),
                                jnp.full((64,), n_edges, jnp.int32)])
    vr = jnp.arange(INST_LEN, dtype=jnp.int32)
    inst = jnp.searchsorted(dst_sorted, vr).astype(jnp.int32)
    src_sorted = src[perm_s]
    outst = jnp.searchsorted(src_sorted, vr).astype(jnp.int32)

    q1, kv1 = _proj1(x, Wq1, jnp.concatenate([Wk1, Wv1], axis=1))
    o1 = _seg_attn_sc(q1, kv1, srcs_pad, inst, 3)[:n_nodes]
    q2, kv2 = _ln_proj(o1, g1, b1, Wq2, jnp.concatenate([Wk2, Wv2], axis=1))
    o2 = _seg_attn_sc(q2, kv2, srcs_pad, inst, 1)[:n_nodes]
    pqA, pqB, pkvA, pkvB = _ln_proj4(
        o2, g2, b2,
        Wq3[:256], Wq3[256:],
        jnp.concatenate([Wk3[:256], Wv3[:256]], axis=1),
        jnp.concatenate([Wk3[256:], Wv3[256:]], axis=1),
    )
    out = _line_attn_sc(pqA, pqB, pkvA, pkvB, srcs_pad, dsts_pad, perm_pad,
                        inst, outst)
    return out[:n_edges]
